# Initial kernel scaffold; baseline (speedup 1.0000x reference)
import jax
import jax.numpy as jnp
from jax.experimental import pallas as pl


def kernel(x, edge_index, mode, eW1, eb1, eW2, eb2, eW3, eb3, nW1, nb1, nW2, nb2, nW3, nb3, dW1, db1, dW2, db2, dW3, db3, dW4, db4):
    raise NotImplementedError("write your pallas kernel here")


# speedup vs baseline: 3.0789x; 3.0789x over previous
import functools

import jax
import jax.numpy as jnp
from jax import lax
from jax.experimental import pallas as pl
from jax.experimental.pallas import tpu as pltpu
from jax.experimental.pallas import tpu_sc as plsc

N = 10000
E = 320000
FEAT = 128
HID = 256

NC = 2
NS = 16
NW = NC * NS
EW = E // NW
GCH = 2000
SCAT_B = 80
SCAT_K = EW // SCAT_B
ZR = N // NS

_SC_PARAMS = pltpu.CompilerParams(needs_layout_passes=False,
                                  use_tc_tiling_on_sc=False)


def _gather_body(x4_hbm, row_hbm, col_hbm, out_hbm, tab_v, ridx_v, cidx_v,
                 obuf_v):
    cid = lax.axis_index("c")
    sid = lax.axis_index("s")
    wid = cid * NS + sid
    pltpu.sync_copy(x4_hbm, tab_v)
    base = wid * EW

    def chunk(j, carry):
        cb = base + j * GCH
        pltpu.sync_copy(row_hbm.at[pl.ds(cb, GCH)], ridx_v)
        pltpu.sync_copy(col_hbm.at[pl.ds(cb, GCH)], cidx_v)

        def group(g, carry2):
            r = ridx_v[pl.ds(g * 16, 16)]
            cc = cidx_v[pl.ds(g * 16, 16)]
            rows = lax.iota(jnp.int32, 16) + g * 16
            for col in range(4):
                cv = jnp.full((16,), col, jnp.int32)
                sv = plsc.load_gather(tab_v, [r, cv])
                dv = plsc.load_gather(tab_v, [cc, cv])
                plsc.store_scatter(obuf_v, [rows, cv], dv - sv)
            return carry2

        lax.fori_loop(0, GCH // 16, group, 0)
        pltpu.sync_copy(obuf_v, out_hbm.at[pl.ds(cb, GCH)])
        return carry

    lax.fori_loop(0, EW // GCH, chunk, 0)


def _scatter_body(ea_hbm, col2d_hbm, zeros_hbm, out_hbm, shared, ebuf_v,
                  ibuf_v):
    cid = lax.axis_index("c")
    sid = lax.axis_index("s")
    pltpu.sync_copy(zeros_hbm.at[pl.ds(sid * ZR, ZR)],
                    shared.at[pl.ds(sid * ZR, ZR)])
    plsc.subcore_barrier()
    base = (cid * NS + sid) * EW
    pltpu.sync_copy(ea_hbm.at[pl.ds(base, EW)], ebuf_v)
    pltpu.sync_copy(col2d_hbm.at[pl.ds(base // SCAT_B, SCAT_K)], ibuf_v)

    def scat(k, carry):
        pltpu.sync_copy(ebuf_v.at[pl.ds(k * SCAT_B, SCAT_B)],
                        shared.at[ibuf_v.at[k]], add=True)
        return carry

    lax.fori_loop(0, SCAT_K, scat, 0)
    plsc.subcore_barrier()
    pltpu.sync_copy(shared.at[pl.ds(sid * ZR, ZR)],
                    out_hbm.at[cid, pl.ds(sid * ZR, ZR)])


@functools.lru_cache(maxsize=None)
def _sc_calls():
    mesh = plsc.VectorSubcoreMesh(core_axis_name="c", subcore_axis_name="s",
                                  num_cores=NC, num_subcores=NS)
    gather_call = pl.kernel(
        _gather_body,
        out_type=jax.ShapeDtypeStruct((E, 8), jnp.float32),
        mesh=mesh,
        compiler_params=_SC_PARAMS,
        scratch_types=[
            pltpu.VMEM((N, 8), jnp.float32),
            pltpu.VMEM((GCH,), jnp.int32),
            pltpu.VMEM((GCH,), jnp.int32),
            pltpu.VMEM((GCH, 8), jnp.float32),
        ],
    )
    scatter_call = pl.kernel(
        _scatter_body,
        out_type=jax.ShapeDtypeStruct((NC, N, 8), jnp.float32),
        mesh=mesh,
        compiler_params=_SC_PARAMS,
        scratch_types=[
            pltpu.VMEM_SHARED((N, 8), jnp.float32),
            pltpu.VMEM((EW, 8), jnp.float32),
            pltpu.VMEM((SCAT_K, SCAT_B), jnp.int32),
        ],
    )
    return gather_call, scatter_call


BLK_E = 2560
BLK_N = 2000


def _edge_mlp_body(e_ref, w1_ref, b1_ref, w2_ref, b2_ref, w3_ref, b3_ref,
                   o_ref):
    e = e_ref[...]
    d3 = e[:, 0:3]
    nrm = jnp.sqrt(jnp.sum(d3 * d3, axis=1, keepdims=True))
    ein = jnp.concatenate(
        [d3, nrm, e[:, 3:4], jnp.zeros((e.shape[0], 3), jnp.float32)], axis=1)
    h = jnp.maximum(
        jnp.dot(ein, w1_ref[...], preferred_element_type=jnp.float32)
        + b1_ref[...], 0.0)
    h = jnp.maximum(
        jnp.dot(h, w2_ref[...], preferred_element_type=jnp.float32)
        + b2_ref[...], 0.0)
    o_ref[...] = (jnp.dot(h, w3_ref[...], preferred_element_type=jnp.float32)
                  + b3_ref[...])


_edge_mlp = pl.pallas_call(
    _edge_mlp_body,
    grid=(E // BLK_E,),
    in_specs=[
        pl.BlockSpec((BLK_E, 8), lambda i: (i, 0)),
        pl.BlockSpec((8, HID), lambda i: (0, 0)),
        pl.BlockSpec((1, HID), lambda i: (0, 0)),
        pl.BlockSpec((HID, HID), lambda i: (0, 0)),
        pl.BlockSpec((1, HID), lambda i: (0, 0)),
        pl.BlockSpec((HID, 8), lambda i: (0, 0)),
        pl.BlockSpec((1, 8), lambda i: (0, 0)),
    ],
    out_specs=pl.BlockSpec((BLK_E, 8), lambda i: (i, 0)),
    out_shape=jax.ShapeDtypeStruct((E, 8), jnp.float32),
    compiler_params=pltpu.CompilerParams(dimension_semantics=("parallel",)),
)


def _node_mlp_body(agg_ref, x_ref, a_ref, bmat_ref, b1_ref, w2_ref, b2_ref,
                   w3_ref, b3_ref, xo_ref, x4o_ref):
    agg2 = agg_ref[...]
    agg = agg2[0] + agg2[1]
    cnt = agg[:, 5:6]
    inv = 1.0 / jnp.maximum(cnt, 1.0)
    m = agg * inv
    x = x_ref[...]
    h = jnp.maximum(
        jnp.dot(m, a_ref[...], preferred_element_type=jnp.float32)
        + jnp.dot(x, bmat_ref[...], preferred_element_type=jnp.float32)
        + b1_ref[...], 0.0)
    h = jnp.maximum(
        jnp.dot(h, w2_ref[...], preferred_element_type=jnp.float32)
        + b2_ref[...], 0.0)
    res = (jnp.dot(h, w3_ref[...], preferred_element_type=jnp.float32)
           + b3_ref[...])
    xn = x + jnp.maximum(res, 0.0)
    xo_ref[...] = xn
    x4o_ref[...] = jnp.concatenate(
        [xn[:, 0:3], xn[:, FEAT - 1:FEAT],
         jnp.zeros((xn.shape[0], 4), jnp.float32)], axis=1)


_node_mlp = pl.pallas_call(
    _node_mlp_body,
    grid=(N // BLK_N,),
    in_specs=[
        pl.BlockSpec((NC, BLK_N, 8), lambda i: (0, i, 0)),
        pl.BlockSpec((BLK_N, FEAT), lambda i: (i, 0)),
        pl.BlockSpec((8, HID), lambda i: (0, 0)),
        pl.BlockSpec((FEAT, HID), lambda i: (0, 0)),
        pl.BlockSpec((1, HID), lambda i: (0, 0)),
        pl.BlockSpec((HID, HID), lambda i: (0, 0)),
        pl.BlockSpec((1, HID), lambda i: (0, 0)),
        pl.BlockSpec((HID, FEAT), lambda i: (0, 0)),
        pl.BlockSpec((1, FEAT), lambda i: (0, 0)),
    ],
    out_specs=[
        pl.BlockSpec((BLK_N, FEAT), lambda i: (i, 0)),
        pl.BlockSpec((BLK_N, 8), lambda i: (i, 0)),
    ],
    out_shape=[
        jax.ShapeDtypeStruct((N, FEAT), jnp.float32),
        jax.ShapeDtypeStruct((N, 8), jnp.float32),
    ],
    compiler_params=pltpu.CompilerParams(dimension_semantics=("parallel",)),
)


def _decoder_body(x_ref, w1_ref, b1_ref, w2_ref, b2_ref, w3_ref, b3_ref,
                  w4_ref, b4_ref, o_ref):
    h = jnp.maximum(
        jnp.dot(x_ref[...], w1_ref[...], preferred_element_type=jnp.float32)
        + b1_ref[...], 0.0)
    h = jnp.maximum(
        jnp.dot(h, w2_ref[...], preferred_element_type=jnp.float32)
        + b2_ref[...], 0.0)
    h = jnp.maximum(
        jnp.dot(h, w3_ref[...], preferred_element_type=jnp.float32)
        + b3_ref[...], 0.0)
    o_ref[...] = (jnp.dot(h, w4_ref[...], preferred_element_type=jnp.float32)
                  + b4_ref[...])


_decoder = pl.pallas_call(
    _decoder_body,
    grid=(N // BLK_N,),
    in_specs=[
        pl.BlockSpec((BLK_N, FEAT), lambda i: (i, 0)),
        pl.BlockSpec((FEAT, HID), lambda i: (0, 0)),
        pl.BlockSpec((1, HID), lambda i: (0, 0)),
        pl.BlockSpec((HID, HID), lambda i: (0, 0)),
        pl.BlockSpec((1, HID), lambda i: (0, 0)),
        pl.BlockSpec((HID, HID), lambda i: (0, 0)),
        pl.BlockSpec((1, HID), lambda i: (0, 0)),
        pl.BlockSpec((HID, 8), lambda i: (0, 0)),
        pl.BlockSpec((1, 8), lambda i: (0, 0)),
    ],
    out_specs=pl.BlockSpec((BLK_N, 8), lambda i: (i, 0)),
    out_shape=jax.ShapeDtypeStruct((N, 8), jnp.float32),
    compiler_params=pltpu.CompilerParams(dimension_semantics=("parallel",)),
)


def kernel(x, edge_index, mode, eW1, eb1, eW2, eb2, eW3, eb3, nW1, nb1, nW2,
           nb2, nW3, nb3, dW1, db1, dW2, db2, dW3, db3, dW4, db4):
    del mode
    row = edge_index[0]
    col = edge_index[1]
    col2d = col.reshape(E // SCAT_B, SCAT_B)
    zeros_n8 = jnp.zeros((N, 8), jnp.float32)

    eW1p = jnp.zeros((8, HID), jnp.float32).at[:5].set(eW1)
    eb1r = eb1.reshape(1, HID)
    eb2r = eb2.reshape(1, HID)
    eW3p = jnp.zeros((HID, 8), jnp.float32).at[:, :5].set(eW3)
    eb3p = jnp.zeros((1, 8), jnp.float32).at[0, :5].set(eb3).at[0, 5].set(1.0)

    nA = jnp.zeros((8, HID), jnp.float32).at[:5].set(nW1[:5])
    nB = jnp.zeros((FEAT, HID), jnp.float32).at[3:].set(nW1[5:])
    nb1r = nb1.reshape(1, HID)
    nb2r = nb2.reshape(1, HID)
    nb3r = nb3.reshape(1, FEAT)

    db1r = db1.reshape(1, HID)
    db2r = db2.reshape(1, HID)
    db3r = db3.reshape(1, HID)
    dW4p = jnp.zeros((HID, 8), jnp.float32).at[:, :3].set(dW4)
    db4p = jnp.zeros((1, 8), jnp.float32).at[0, :3].set(db4)

    x4 = jnp.concatenate(
        [x[:, :3], x[:, FEAT - 1:], jnp.zeros((N, 4), jnp.float32)], axis=1)

    gather_call, scatter_call = _sc_calls()
    for _ in range(3):
        epre = gather_call(x4, row, col)
        eattr = _edge_mlp(epre, eW1p, eb1r, eW2, eb2r, eW3p, eb3p)
        agg2 = scatter_call(eattr, col2d, zeros_n8)
        x, x4 = _node_mlp(agg2, x, nA, nB, nb1r, nW2, nb2r, nW3, nb3r)

    out8 = _decoder(x, dW1, db1r, dW2, db2r, dW3, db3r, dW4p, db4p)
    return out8[:, :3]


# speedup vs baseline: 5.8241x; 1.8916x over previous
import functools

import jax
import jax.numpy as jnp
from jax import lax
from jax.experimental import pallas as pl
from jax.experimental.pallas import tpu as pltpu
from jax.experimental.pallas import tpu_sc as plsc

N = 10000
E = 320000
FEAT = 128
HID = 256

NC = 2
NS = 16
NW = NC * NS
EW = E // NW
GCH = 2000
SCAT_B = 80
SCAT_K = EW // SCAT_B
ZR = N // NS

_SC_PARAMS = pltpu.CompilerParams(needs_layout_passes=False,
                                  use_tc_tiling_on_sc=False)


def _gather_body(x4_hbm, row_hbm, col_hbm, out_hbm, tab_v, ridx_v, cidx_v,
                 obuf_v):
    cid = lax.axis_index("c")
    sid = lax.axis_index("s")
    wid = cid * NS + sid
    pltpu.sync_copy(x4_hbm, tab_v)
    base = wid * EW

    def chunk(j, carry):
        cb = base + j * GCH
        pltpu.sync_copy(row_hbm.at[pl.ds(cb, GCH)], ridx_v)
        pltpu.sync_copy(col_hbm.at[pl.ds(cb, GCH)], cidx_v)

        def group(g, carry2):
            r = ridx_v[pl.ds(g * 16, 16)]
            cc = cidx_v[pl.ds(g * 16, 16)]
            rows = lax.iota(jnp.int32, 16) + g * 16
            d = []
            for col in range(4):
                cv = jnp.full((16,), col, jnp.int32)
                sv = plsc.load_gather(tab_v, [r, cv])
                dv = plsc.load_gather(tab_v, [cc, cv])
                d.append(dv - sv)
                plsc.store_scatter(obuf_v, [rows, cv], d[col])
            nsq = d[0] * d[0] + d[1] * d[1] + d[2] * d[2]
            i = plsc.bitcast(nsq, jnp.int32)
            i = 0x5F3759DF - lax.shift_right_logical(i, 1)
            y = plsc.bitcast(i, jnp.float32)
            for _ in range(3):
                y = y * (1.5 - 0.5 * nsq * y * y)
            nrm = jnp.where(nsq > 0.0, nsq * y, 0.0)
            plsc.store_scatter(obuf_v, [rows, jnp.full((16,), 4, jnp.int32)],
                               nrm)
            return carry2

        lax.fori_loop(0, GCH // 16, group, 0)
        pltpu.sync_copy(obuf_v, out_hbm.at[pl.ds(cb, GCH)])
        return carry

    lax.fori_loop(0, EW // GCH, chunk, 0)


def _scatter_body(ea_hbm, col2d_hbm, zeros_hbm, out_hbm, shared, ebuf_v,
                  ibuf_v):
    cid = lax.axis_index("c")
    sid = lax.axis_index("s")
    pltpu.sync_copy(zeros_hbm.at[pl.ds(sid * ZR, ZR)],
                    shared.at[pl.ds(sid * ZR, ZR)])
    plsc.subcore_barrier()
    base = (cid * NS + sid) * EW
    pltpu.sync_copy(ea_hbm.at[pl.ds(base, EW)], ebuf_v)
    pltpu.sync_copy(col2d_hbm.at[pl.ds(base // SCAT_B, SCAT_K)], ibuf_v)

    def scat(k, carry):
        pltpu.sync_copy(ebuf_v.at[pl.ds(k * SCAT_B, SCAT_B)],
                        shared.at[ibuf_v.at[k]], add=True)
        return carry

    lax.fori_loop(0, SCAT_K, scat, 0)
    plsc.subcore_barrier()
    pltpu.sync_copy(shared.at[pl.ds(sid * ZR, ZR)],
                    out_hbm.at[cid, pl.ds(sid * ZR, ZR)])


@functools.lru_cache(maxsize=None)
def _sc_calls():
    mesh = plsc.VectorSubcoreMesh(core_axis_name="c", subcore_axis_name="s",
                                  num_cores=NC, num_subcores=NS)
    gather_call = pl.kernel(
        _gather_body,
        out_type=jax.ShapeDtypeStruct((E, 8), jnp.float32),
        mesh=mesh,
        compiler_params=_SC_PARAMS,
        scratch_types=[
            pltpu.VMEM((N, 8), jnp.float32),
            pltpu.VMEM((GCH,), jnp.int32),
            pltpu.VMEM((GCH,), jnp.int32),
            pltpu.VMEM((GCH, 8), jnp.float32),
        ],
    )
    scatter_call = pl.kernel(
        _scatter_body,
        out_type=jax.ShapeDtypeStruct((NC, N, 8), jnp.float32),
        mesh=mesh,
        compiler_params=_SC_PARAMS,
        scratch_types=[
            pltpu.VMEM_SHARED((N, 8), jnp.float32),
            pltpu.VMEM((EW, 8), jnp.float32),
            pltpu.VMEM((SCAT_K, SCAT_B), jnp.int32),
        ],
    )
    return gather_call, scatter_call


BLK_E = 4000
BLK_N = 2000


def _edge_mlp_body(e_ref, w1_ref, b1_ref, w2_ref, b2_ref, w3_ref,
                   b3_ref, o_ref):
    e = e_ref[...]
    h = jnp.maximum(
        jnp.dot(e.astype(jnp.bfloat16), w1_ref[...],
                preferred_element_type=jnp.float32)
        + b1_ref[...], 0.0)
    h = jnp.maximum(
        jnp.dot(h.astype(jnp.bfloat16), w2_ref[...],
                preferred_element_type=jnp.float32)
        + b2_ref[...], 0.0)
    o_ref[...] = (jnp.dot(h.astype(jnp.bfloat16), w3_ref[...],
                          preferred_element_type=jnp.float32)
                  + b3_ref[...])


_edge_mlp = pl.pallas_call(
    _edge_mlp_body,
    grid=(E // BLK_E,),
    in_specs=[
        pl.BlockSpec((BLK_E, 8), lambda i: (i, 0)),
        pl.BlockSpec((8, HID), lambda i: (0, 0)),
        pl.BlockSpec((1, HID), lambda i: (0, 0)),
        pl.BlockSpec((HID, HID), lambda i: (0, 0)),
        pl.BlockSpec((1, HID), lambda i: (0, 0)),
        pl.BlockSpec((HID, 8), lambda i: (0, 0)),
        pl.BlockSpec((1, 8), lambda i: (0, 0)),
    ],
    out_specs=pl.BlockSpec((BLK_E, 8), lambda i: (i, 0)),
    out_shape=jax.ShapeDtypeStruct((E, 8), jnp.float32),
    compiler_params=pltpu.CompilerParams(dimension_semantics=("parallel",)),
)


def _node_mlp_body(agg_ref, x_ref, a_ref, bmat_ref, b1_ref, w2_ref, b2_ref,
                   w3_ref, b3_ref, xo_ref, x4o_ref):
    agg2 = agg_ref[...]
    agg = agg2[0] + agg2[1]
    cnt = agg[:, 5:6]
    inv = 1.0 / jnp.maximum(cnt, 1.0)
    m = agg * inv
    x = x_ref[...]
    h = jnp.maximum(
        jnp.dot(m, a_ref[...], preferred_element_type=jnp.float32)
        + jnp.dot(x, bmat_ref[...], preferred_element_type=jnp.float32)
        + b1_ref[...], 0.0)
    h = jnp.maximum(
        jnp.dot(h, w2_ref[...], preferred_element_type=jnp.float32)
        + b2_ref[...], 0.0)
    res = (jnp.dot(h, w3_ref[...], preferred_element_type=jnp.float32)
           + b3_ref[...])
    xn = x + jnp.maximum(res, 0.0)
    xo_ref[...] = xn
    x4o_ref[...] = jnp.concatenate(
        [xn[:, 0:3], xn[:, FEAT - 1:FEAT],
         jnp.zeros((xn.shape[0], 4), jnp.float32)], axis=1)


_node_mlp = pl.pallas_call(
    _node_mlp_body,
    grid=(N // BLK_N,),
    in_specs=[
        pl.BlockSpec((NC, BLK_N, 8), lambda i: (0, i, 0)),
        pl.BlockSpec((BLK_N, FEAT), lambda i: (i, 0)),
        pl.BlockSpec((8, HID), lambda i: (0, 0)),
        pl.BlockSpec((FEAT, HID), lambda i: (0, 0)),
        pl.BlockSpec((1, HID), lambda i: (0, 0)),
        pl.BlockSpec((HID, HID), lambda i: (0, 0)),
        pl.BlockSpec((1, HID), lambda i: (0, 0)),
        pl.BlockSpec((HID, FEAT), lambda i: (0, 0)),
        pl.BlockSpec((1, FEAT), lambda i: (0, 0)),
    ],
    out_specs=[
        pl.BlockSpec((BLK_N, FEAT), lambda i: (i, 0)),
        pl.BlockSpec((BLK_N, 8), lambda i: (i, 0)),
    ],
    out_shape=[
        jax.ShapeDtypeStruct((N, FEAT), jnp.float32),
        jax.ShapeDtypeStruct((N, 8), jnp.float32),
    ],
    compiler_params=pltpu.CompilerParams(dimension_semantics=("parallel",)),
)


def _decoder_body(x_ref, w1_ref, b1_ref, w2_ref, b2_ref, w3_ref, b3_ref,
                  w4_ref, b4_ref, o_ref):
    h = jnp.maximum(
        jnp.dot(x_ref[...], w1_ref[...], preferred_element_type=jnp.float32)
        + b1_ref[...], 0.0)
    h = jnp.maximum(
        jnp.dot(h, w2_ref[...], preferred_element_type=jnp.float32)
        + b2_ref[...], 0.0)
    h = jnp.maximum(
        jnp.dot(h, w3_ref[...], preferred_element_type=jnp.float32)
        + b3_ref[...], 0.0)
    o_ref[...] = (jnp.dot(h, w4_ref[...], preferred_element_type=jnp.float32)
                  + b4_ref[...])


_decoder = pl.pallas_call(
    _decoder_body,
    grid=(N // BLK_N,),
    in_specs=[
        pl.BlockSpec((BLK_N, FEAT), lambda i: (i, 0)),
        pl.BlockSpec((FEAT, HID), lambda i: (0, 0)),
        pl.BlockSpec((1, HID), lambda i: (0, 0)),
        pl.BlockSpec((HID, HID), lambda i: (0, 0)),
        pl.BlockSpec((1, HID), lambda i: (0, 0)),
        pl.BlockSpec((HID, HID), lambda i: (0, 0)),
        pl.BlockSpec((1, HID), lambda i: (0, 0)),
        pl.BlockSpec((HID, 8), lambda i: (0, 0)),
        pl.BlockSpec((1, 8), lambda i: (0, 0)),
    ],
    out_specs=pl.BlockSpec((BLK_N, 8), lambda i: (i, 0)),
    out_shape=jax.ShapeDtypeStruct((N, 8), jnp.float32),
    compiler_params=pltpu.CompilerParams(dimension_semantics=("parallel",)),
)


def kernel(x, edge_index, mode, eW1, eb1, eW2, eb2, eW3, eb3, nW1, nb1, nW2,
           nb2, nW3, nb3, dW1, db1, dW2, db2, dW3, db3, dW4, db4):
    del mode
    row = edge_index[0]
    col = edge_index[1]
    col2d = col.reshape(E // SCAT_B, SCAT_B)
    zeros_n8 = jnp.zeros((N, 8), jnp.float32)

    eW1p = (jnp.zeros((8, HID), jnp.float32)
            .at[:3].set(eW1[:3]).at[3].set(eW1[4]).at[4].set(eW1[3])
            ).astype(jnp.bfloat16)
    eb1r = eb1.reshape(1, HID)
    eW2b = eW2.astype(jnp.bfloat16)
    eb2r = eb2.reshape(1, HID)
    eW3p = (jnp.zeros((HID, 8), jnp.float32).at[:, :5].set(eW3)
            ).astype(jnp.bfloat16)
    eb3p = jnp.zeros((1, 8), jnp.float32).at[0, :5].set(eb3).at[0, 5].set(1.0)

    nA = jnp.zeros((8, HID), jnp.float32).at[:5].set(nW1[:5])
    nB = jnp.zeros((FEAT, HID), jnp.float32).at[3:].set(nW1[5:])
    nb1r = nb1.reshape(1, HID)
    nb2r = nb2.reshape(1, HID)
    nb3r = nb3.reshape(1, FEAT)

    db1r = db1.reshape(1, HID)
    db2r = db2.reshape(1, HID)
    db3r = db3.reshape(1, HID)
    dW4p = jnp.zeros((HID, 8), jnp.float32).at[:, :3].set(dW4)
    db4p = jnp.zeros((1, 8), jnp.float32).at[0, :3].set(db4)

    x4 = jnp.concatenate(
        [x[:, :3], x[:, FEAT - 1:], jnp.zeros((N, 4), jnp.float32)], axis=1)

    gather_call, scatter_call = _sc_calls()
    for _ in range(3):
        epre = gather_call(x4, row, col)
        eattr = _edge_mlp(epre, eW1p, eb1r, eW2b, eb2r, eW3p, eb3p)
        agg2 = scatter_call(eattr, col2d, zeros_n8)
        x, x4 = _node_mlp(agg2, x, nA, nB, nb1r, nW2, nb2r, nW3, nb3r)

    out8 = _decoder(x, dW1, db1r, dW2, db2r, dW3, db3r, dW4p, db4p)
    return out8[:, :3]


# speedup vs baseline: 9.6138x; 1.6507x over previous
import functools

import jax
import jax.numpy as jnp
from jax import lax
from jax.experimental import pallas as pl
from jax.experimental.pallas import tpu as pltpu
from jax.experimental.pallas import tpu_sc as plsc

N = 10000
E = 320000
FEAT = 128
HID = 256

NC = 2
NS = 16
NW = NC * NS
PAN = 128
NPAN = E // PAN
PPT = NPAN // NW
PREM = NPAN - PPT * NW
PCH = 26
ZR = N // NS

_SC_PARAMS = pltpu.CompilerParams(needs_layout_passes=False,
                                  use_tc_tiling_on_sc=False)


def _gather_body(x4_hbm, row_hbm, col_hbm, out_hbm, tab_v, ridx_v, cidx_v,
                 obuf_v):
    cid = lax.axis_index("c")
    sid = lax.axis_index("s")
    wid = cid * NS + sid
    pltpu.sync_copy(x4_hbm, tab_v)

    def do_panels(pbase, npan):
        ne = npan * PAN
        eb = pbase * PAN
        pltpu.sync_copy(row_hbm.at[pl.ds(eb, ne)], ridx_v.at[pl.ds(0, ne)])
        pltpu.sync_copy(col_hbm.at[pl.ds(eb, ne)], cidx_v.at[pl.ds(0, ne)])

        def panel(jj, carry):
            for g in range(8):
                o = jj * PAN + g * 16
                r = ridx_v[pl.ds(o, 16)]
                cc = cidx_v[pl.ds(o, 16)]
                d = []
                for col in range(4):
                    cv = jnp.full((16,), col, jnp.int32)
                    sv = plsc.load_gather(tab_v, [r, cv])
                    dv = plsc.load_gather(tab_v, [cc, cv])
                    d.append(dv - sv)
                    obuf_v[jj, col, pl.ds(g * 16, 16)] = d[col]
                nsq = d[0] * d[0] + d[1] * d[1] + d[2] * d[2]
                i = plsc.bitcast(nsq, jnp.int32)
                i = 0x5F3759DF - lax.shift_right_logical(i, 1)
                y = plsc.bitcast(i, jnp.float32)
                for _ in range(3):
                    y = y * (1.5 - 0.5 * nsq * y * y)
                nrm = jnp.where(nsq > 0.0, nsq * y, 0.0)
                obuf_v[jj, 4, pl.ds(g * 16, 16)] = nrm
            return carry

        lax.fori_loop(0, npan, panel, 0)
        pltpu.sync_copy(obuf_v.at[pl.ds(0, npan)],
                        out_hbm.at[pl.ds(pbase, npan)])

    for ch in range(PPT // PCH):
        do_panels(wid * PPT + ch * PCH, PCH)

    @pl.when(wid < PREM)
    def _():
        do_panels(NW * PPT + wid, 1)


def _scatter_body(ea_hbm, col2d_hbm, zeros_hbm, out_hbm, shared, ebuf_v,
                  ibuf_v, rbuf_v):
    cid = lax.axis_index("c")
    sid = lax.axis_index("s")
    wid = cid * NS + sid
    pltpu.sync_copy(zeros_hbm.at[pl.ds(sid * ZR, ZR)],
                    shared.at[pl.ds(sid * ZR, ZR)])
    zero16 = jnp.zeros((16,), jnp.float32)
    for g in range(8):
        rows = lax.iota(jnp.int32, 16) + g * 16
        plsc.store_scatter(rbuf_v, [rows, jnp.full((16,), 6, jnp.int32)],
                           zero16)
        plsc.store_scatter(rbuf_v, [rows, jnp.full((16,), 7, jnp.int32)],
                           zero16)
    plsc.subcore_barrier()

    def do_panels(pbase, npan):
        pltpu.sync_copy(ea_hbm.at[pl.ds(pbase, npan)],
                        ebuf_v.at[pl.ds(0, npan)])
        pltpu.sync_copy(col2d_hbm.at[pl.ds(pbase, npan)],
                        ibuf_v.at[pl.ds(0, npan)])

        def panel(jj, carry):
            for g in range(8):
                rows = lax.iota(jnp.int32, 16) + g * 16
                for c in range(6):
                    v = ebuf_v[jj, c, pl.ds(g * 16, 16)]
                    plsc.store_scatter(rbuf_v, [rows,
                                                jnp.full((16,), c, jnp.int32)],
                                       v)
            pltpu.sync_copy(rbuf_v, shared.at[ibuf_v.at[jj]], add=True)
            return carry

        lax.fori_loop(0, npan, panel, 0)

    for ch in range(PPT // PCH):
        do_panels(wid * PPT + ch * PCH, PCH)

    @pl.when(wid < PREM)
    def _():
        do_panels(NW * PPT + wid, 1)

    plsc.subcore_barrier()
    pltpu.sync_copy(shared.at[pl.ds(sid * ZR, ZR)],
                    out_hbm.at[cid, pl.ds(sid * ZR, ZR)])


@functools.lru_cache(maxsize=None)
def _sc_calls():
    mesh = plsc.VectorSubcoreMesh(core_axis_name="c", subcore_axis_name="s",
                                  num_cores=NC, num_subcores=NS)
    gather_call = pl.kernel(
        _gather_body,
        out_type=jax.ShapeDtypeStruct((NPAN, 8, PAN), jnp.float32),
        mesh=mesh,
        compiler_params=_SC_PARAMS,
        scratch_types=[
            pltpu.VMEM((N, 8), jnp.float32),
            pltpu.VMEM((PCH * PAN,), jnp.int32),
            pltpu.VMEM((PCH * PAN,), jnp.int32),
            pltpu.VMEM((PCH, 8, PAN), jnp.float32),
        ],
    )
    scatter_call = pl.kernel(
        _scatter_body,
        out_type=jax.ShapeDtypeStruct((NC, N, 8), jnp.float32),
        mesh=mesh,
        compiler_params=_SC_PARAMS,
        scratch_types=[
            pltpu.VMEM_SHARED((N, 8), jnp.float32),
            pltpu.VMEM((PCH, 8, PAN), jnp.float32),
            pltpu.VMEM((PCH, PAN), jnp.int32),
            pltpu.VMEM((PAN, 8), jnp.float32),
        ],
    )
    return gather_call, scatter_call


BP = 25
BLK_N = 2000


def _edge_mlp_body(e_ref, w1_ref, b1_ref, w2_ref, b2_ref, w3_ref,
                   b3_ref, o_ref):
    e3 = e_ref[...]
    e = jnp.transpose(e3, (0, 2, 1)).reshape(BP * PAN, 8)
    h = jnp.maximum(
        jnp.dot(e.astype(jnp.bfloat16), w1_ref[...],
                preferred_element_type=jnp.float32)
        + b1_ref[...], 0.0)
    h = jnp.maximum(
        jnp.dot(h.astype(jnp.bfloat16), w2_ref[...],
                preferred_element_type=jnp.float32)
        + b2_ref[...], 0.0)
    o = (jnp.dot(h.astype(jnp.bfloat16), w3_ref[...],
                 preferred_element_type=jnp.float32)
         + b3_ref[...])
    o_ref[...] = jnp.transpose(o.reshape(BP, PAN, 8), (0, 2, 1))


_edge_mlp = pl.pallas_call(
    _edge_mlp_body,
    grid=(NPAN // BP,),
    in_specs=[
        pl.BlockSpec((BP, 8, PAN), lambda i: (i, 0, 0)),
        pl.BlockSpec((8, HID), lambda i: (0, 0)),
        pl.BlockSpec((1, HID), lambda i: (0, 0)),
        pl.BlockSpec((HID, HID), lambda i: (0, 0)),
        pl.BlockSpec((1, HID), lambda i: (0, 0)),
        pl.BlockSpec((HID, 8), lambda i: (0, 0)),
        pl.BlockSpec((1, 8), lambda i: (0, 0)),
    ],
    out_specs=pl.BlockSpec((BP, 8, PAN), lambda i: (i, 0, 0)),
    out_shape=jax.ShapeDtypeStruct((NPAN, 8, PAN), jnp.float32),
    compiler_params=pltpu.CompilerParams(dimension_semantics=("parallel",)),
)


def _node_mlp_body(agg_ref, x_ref, a_ref, bmat_ref, b1_ref, w2_ref, b2_ref,
                   w3_ref, b3_ref, xo_ref, x4o_ref):
    agg2 = agg_ref[...]
    agg = agg2[0] + agg2[1]
    cnt = agg[:, 5:6]
    inv = 1.0 / jnp.maximum(cnt, 1.0)
    m = agg * inv
    x = x_ref[...]
    h = jnp.maximum(
        jnp.dot(m, a_ref[...], preferred_element_type=jnp.float32)
        + jnp.dot(x, bmat_ref[...], preferred_element_type=jnp.float32)
        + b1_ref[...], 0.0)
    h = jnp.maximum(
        jnp.dot(h, w2_ref[...], preferred_element_type=jnp.float32)
        + b2_ref[...], 0.0)
    res = (jnp.dot(h, w3_ref[...], preferred_element_type=jnp.float32)
           + b3_ref[...])
    xn = x + jnp.maximum(res, 0.0)
    xo_ref[...] = xn
    x4o_ref[...] = jnp.concatenate(
        [xn[:, 0:3], xn[:, FEAT - 1:FEAT],
         jnp.zeros((xn.shape[0], 4), jnp.float32)], axis=1)


_node_mlp = pl.pallas_call(
    _node_mlp_body,
    grid=(N // BLK_N,),
    in_specs=[
        pl.BlockSpec((NC, BLK_N, 8), lambda i: (0, i, 0)),
        pl.BlockSpec((BLK_N, FEAT), lambda i: (i, 0)),
        pl.BlockSpec((8, HID), lambda i: (0, 0)),
        pl.BlockSpec((FEAT, HID), lambda i: (0, 0)),
        pl.BlockSpec((1, HID), lambda i: (0, 0)),
        pl.BlockSpec((HID, HID), lambda i: (0, 0)),
        pl.BlockSpec((1, HID), lambda i: (0, 0)),
        pl.BlockSpec((HID, FEAT), lambda i: (0, 0)),
        pl.BlockSpec((1, FEAT), lambda i: (0, 0)),
    ],
    out_specs=[
        pl.BlockSpec((BLK_N, FEAT), lambda i: (i, 0)),
        pl.BlockSpec((BLK_N, 8), lambda i: (i, 0)),
    ],
    out_shape=[
        jax.ShapeDtypeStruct((N, FEAT), jnp.float32),
        jax.ShapeDtypeStruct((N, 8), jnp.float32),
    ],
    compiler_params=pltpu.CompilerParams(dimension_semantics=("parallel",)),
)


def _decoder_body(x_ref, w1_ref, b1_ref, w2_ref, b2_ref, w3_ref, b3_ref,
                  w4_ref, b4_ref, o_ref):
    h = jnp.maximum(
        jnp.dot(x_ref[...], w1_ref[...], preferred_element_type=jnp.float32)
        + b1_ref[...], 0.0)
    h = jnp.maximum(
        jnp.dot(h, w2_ref[...], preferred_element_type=jnp.float32)
        + b2_ref[...], 0.0)
    h = jnp.maximum(
        jnp.dot(h, w3_ref[...], preferred_element_type=jnp.float32)
        + b3_ref[...], 0.0)
    o_ref[...] = (jnp.dot(h, w4_ref[...], preferred_element_type=jnp.float32)
                  + b4_ref[...])


_decoder = pl.pallas_call(
    _decoder_body,
    grid=(N // BLK_N,),
    in_specs=[
        pl.BlockSpec((BLK_N, FEAT), lambda i: (i, 0)),
        pl.BlockSpec((FEAT, HID), lambda i: (0, 0)),
        pl.BlockSpec((1, HID), lambda i: (0, 0)),
        pl.BlockSpec((HID, HID), lambda i: (0, 0)),
        pl.BlockSpec((1, HID), lambda i: (0, 0)),
        pl.BlockSpec((HID, HID), lambda i: (0, 0)),
        pl.BlockSpec((1, HID), lambda i: (0, 0)),
        pl.BlockSpec((HID, 8), lambda i: (0, 0)),
        pl.BlockSpec((1, 8), lambda i: (0, 0)),
    ],
    out_specs=pl.BlockSpec((BLK_N, 8), lambda i: (i, 0)),
    out_shape=jax.ShapeDtypeStruct((N, 8), jnp.float32),
    compiler_params=pltpu.CompilerParams(dimension_semantics=("parallel",)),
)


def kernel(x, edge_index, mode, eW1, eb1, eW2, eb2, eW3, eb3, nW1, nb1, nW2,
           nb2, nW3, nb3, dW1, db1, dW2, db2, dW3, db3, dW4, db4):
    del mode
    row = edge_index[0]
    col = edge_index[1]
    col2d = col.reshape(NPAN, PAN)
    zeros_n8 = jnp.zeros((N, 8), jnp.float32)

    eW1p = (jnp.zeros((8, HID), jnp.float32)
            .at[:3].set(eW1[:3]).at[3].set(eW1[4]).at[4].set(eW1[3])
            ).astype(jnp.bfloat16)
    eb1r = eb1.reshape(1, HID)
    eW2b = eW2.astype(jnp.bfloat16)
    eb2r = eb2.reshape(1, HID)
    eW3p = (jnp.zeros((HID, 8), jnp.float32).at[:, :5].set(eW3)
            ).astype(jnp.bfloat16)
    eb3p = jnp.zeros((1, 8), jnp.float32).at[0, :5].set(eb3).at[0, 5].set(1.0)

    nA = jnp.zeros((8, HID), jnp.float32).at[:5].set(nW1[:5])
    nB = jnp.zeros((FEAT, HID), jnp.float32).at[3:].set(nW1[5:])
    nb1r = nb1.reshape(1, HID)
    nb2r = nb2.reshape(1, HID)
    nb3r = nb3.reshape(1, FEAT)

    db1r = db1.reshape(1, HID)
    db2r = db2.reshape(1, HID)
    db3r = db3.reshape(1, HID)
    dW4p = jnp.zeros((HID, 8), jnp.float32).at[:, :3].set(dW4)
    db4p = jnp.zeros((1, 8), jnp.float32).at[0, :3].set(db4)

    x4 = jnp.concatenate(
        [x[:, :3], x[:, FEAT - 1:], jnp.zeros((N, 4), jnp.float32)], axis=1)

    gather_call, scatter_call = _sc_calls()
    for _ in range(3):
        epre = gather_call(x4, row, col)
        eattr = _edge_mlp(epre, eW1p, eb1r, eW2b, eb2r, eW3p, eb3p)
        agg2 = scatter_call(eattr, col2d, zeros_n8)
        x, x4 = _node_mlp(agg2, x, nA, nB, nb1r, nW2, nb2r, nW3, nb3r)

    out8 = _decoder(x, dW1, db1r, dW2, db2r, dW3, db3r, dW4p, db4p)
    return out8[:, :3]


# speedup vs baseline: 10.2469x; 1.0659x over previous
import functools

import jax
import jax.numpy as jnp
from jax import lax
from jax.experimental import pallas as pl
from jax.experimental.pallas import tpu as pltpu
from jax.experimental.pallas import tpu_sc as plsc

N = 10000
E = 320000
FEAT = 128
HID = 256

NC = 2
NS = 16
NW = NC * NS
PAN = 128
NPAN = E // PAN
PPT = NPAN // NW
PREM = NPAN - PPT * NW
PCH = 26
ZR = N // NS

_SC_PARAMS = pltpu.CompilerParams(needs_layout_passes=False,
                                  use_tc_tiling_on_sc=False)


def _gather_body(x4_hbm, row_hbm, col_hbm, out_hbm, tab_v, ridx_v, cidx_v,
                 obuf_v):
    cid = lax.axis_index("c")
    sid = lax.axis_index("s")
    wid = cid * NS + sid
    pltpu.sync_copy(x4_hbm, tab_v)

    def do_panels(pbase, npan):
        ne = npan * PAN
        eb = pbase * PAN
        pltpu.sync_copy(row_hbm.at[pl.ds(eb, ne)], ridx_v.at[pl.ds(0, ne)])
        pltpu.sync_copy(col_hbm.at[pl.ds(eb, ne)], cidx_v.at[pl.ds(0, ne)])

        def panel(jj, carry):
            for g in range(8):
                o = jj * PAN + g * 16
                r = ridx_v[pl.ds(o, 16)]
                cc = cidx_v[pl.ds(o, 16)]
                d = []
                for col in range(4):
                    cv = jnp.full((16,), col, jnp.int32)
                    sv = plsc.load_gather(tab_v, [r, cv])
                    dv = plsc.load_gather(tab_v, [cc, cv])
                    d.append(dv - sv)
                    obuf_v[jj, col, pl.ds(g * 16, 16)] = d[col]
                nsq = d[0] * d[0] + d[1] * d[1] + d[2] * d[2]
                i = plsc.bitcast(nsq, jnp.int32)
                i = 0x5F3759DF - lax.shift_right_logical(i, 1)
                y = plsc.bitcast(i, jnp.float32)
                for _ in range(3):
                    y = y * (1.5 - 0.5 * nsq * y * y)
                nrm = jnp.where(nsq > 0.0, nsq * y, 0.0)
                obuf_v[jj, 4, pl.ds(g * 16, 16)] = nrm
                obuf_v[jj, 5, pl.ds(g * 16, 16)] = jnp.full((16,), 1.0,
                                                            jnp.float32)
            return carry

        lax.fori_loop(0, npan, panel, 0)
        pltpu.sync_copy(obuf_v.at[pl.ds(0, npan)],
                        out_hbm.at[pl.ds(pbase, npan)])

    for ch in range(PPT // PCH):
        do_panels(wid * PPT + ch * PCH, PCH)

    @pl.when(wid < PREM)
    def _():
        do_panels(NW * PPT + wid, 1)


def _scatter_body(ea_hbm, col2d_hbm, zeros_hbm, out_hbm, shared, ebuf_v,
                  ibuf_v, rbuf_v):
    cid = lax.axis_index("c")
    sid = lax.axis_index("s")
    wid = cid * NS + sid
    pltpu.sync_copy(zeros_hbm.at[pl.ds(sid * ZR, ZR)],
                    shared.at[pl.ds(sid * ZR, ZR)])
    zero16 = jnp.zeros((16,), jnp.float32)
    for g in range(8):
        rows = lax.iota(jnp.int32, 16) + g * 16
        plsc.store_scatter(rbuf_v, [rows, jnp.full((16,), 6, jnp.int32)],
                           zero16)
        plsc.store_scatter(rbuf_v, [rows, jnp.full((16,), 7, jnp.int32)],
                           zero16)
    plsc.subcore_barrier()

    def do_panels(pbase, npan):
        pltpu.sync_copy(ea_hbm.at[pl.ds(pbase, npan)],
                        ebuf_v.at[pl.ds(0, npan)])
        pltpu.sync_copy(col2d_hbm.at[pl.ds(pbase, npan)],
                        ibuf_v.at[pl.ds(0, npan)])

        def panel(jj, carry):
            for g in range(8):
                rows = lax.iota(jnp.int32, 16) + g * 16
                for c in range(6):
                    v = ebuf_v[jj, c, pl.ds(g * 16, 16)]
                    plsc.store_scatter(rbuf_v, [rows,
                                                jnp.full((16,), c, jnp.int32)],
                                       v)
            pltpu.sync_copy(rbuf_v, shared.at[ibuf_v.at[jj]], add=True)
            return carry

        lax.fori_loop(0, npan, panel, 0)

    for ch in range(PPT // PCH):
        do_panels(wid * PPT + ch * PCH, PCH)

    @pl.when(wid < PREM)
    def _():
        do_panels(NW * PPT + wid, 1)

    plsc.subcore_barrier()
    pltpu.sync_copy(shared.at[pl.ds(sid * ZR, ZR)],
                    out_hbm.at[cid, pl.ds(sid * ZR, ZR)])


@functools.lru_cache(maxsize=None)
def _sc_calls():
    mesh = plsc.VectorSubcoreMesh(core_axis_name="c", subcore_axis_name="s",
                                  num_cores=NC, num_subcores=NS)
    gather_call = pl.kernel(
        _gather_body,
        out_type=jax.ShapeDtypeStruct((NPAN, 8, PAN), jnp.float32),
        mesh=mesh,
        compiler_params=_SC_PARAMS,
        scratch_types=[
            pltpu.VMEM((N, 8), jnp.float32),
            pltpu.VMEM((PCH * PAN,), jnp.int32),
            pltpu.VMEM((PCH * PAN,), jnp.int32),
            pltpu.VMEM((PCH, 8, PAN), jnp.float32),
        ],
    )
    scatter_call = pl.kernel(
        _scatter_body,
        out_type=jax.ShapeDtypeStruct((NC, N, 8), jnp.float32),
        mesh=mesh,
        compiler_params=_SC_PARAMS,
        scratch_types=[
            pltpu.VMEM_SHARED((N, 8), jnp.float32),
            pltpu.VMEM((PCH, 8, PAN), jnp.float32),
            pltpu.VMEM((PCH, PAN), jnp.int32),
            pltpu.VMEM((PAN, 8), jnp.float32),
        ],
    )
    return gather_call, scatter_call


BP = 50
BLK_N = 2000


def _edge_mlp_body(e_ref, w1_ref, w2_ref, b2_ref, w3_ref,
                   b3_ref, o_ref):
    e3 = e_ref[...]
    e = jnp.transpose(e3, (0, 2, 1)).reshape(BP * PAN, 8)
    h = jnp.maximum(
        jnp.dot(e.astype(jnp.bfloat16), w1_ref[...],
                preferred_element_type=jnp.float32), 0.0)
    h = jnp.maximum(
        jnp.dot(h.astype(jnp.bfloat16), w2_ref[...],
                preferred_element_type=jnp.float32)
        + b2_ref[...], 0.0)
    o = (jnp.dot(h.astype(jnp.bfloat16), w3_ref[...],
                 preferred_element_type=jnp.float32)
         + b3_ref[...])
    o_ref[...] = jnp.transpose(o.reshape(BP, PAN, 8), (0, 2, 1))


_edge_mlp = pl.pallas_call(
    _edge_mlp_body,
    grid=(NPAN // BP,),
    in_specs=[
        pl.BlockSpec((BP, 8, PAN), lambda i: (i, 0, 0)),
        pl.BlockSpec((8, HID), lambda i: (0, 0)),
        pl.BlockSpec((HID, HID), lambda i: (0, 0)),
        pl.BlockSpec((1, HID), lambda i: (0, 0)),
        pl.BlockSpec((HID, 8), lambda i: (0, 0)),
        pl.BlockSpec((1, 8), lambda i: (0, 0)),
    ],
    out_specs=pl.BlockSpec((BP, 8, PAN), lambda i: (i, 0, 0)),
    out_shape=jax.ShapeDtypeStruct((NPAN, 8, PAN), jnp.float32),
    compiler_params=pltpu.CompilerParams(dimension_semantics=("parallel",)),
)


def _node_mlp_body(agg_ref, x_ref, a_ref, bmat_ref, b1_ref, w2_ref, b2_ref,
                   w3_ref, b3_ref, xo_ref, x4o_ref):
    agg2 = agg_ref[...]
    agg = agg2[0] + agg2[1]
    cnt = agg[:, 5:6]
    inv = 1.0 / jnp.maximum(cnt, 1.0)
    m = agg * inv
    x = x_ref[...]
    h = jnp.maximum(
        jnp.dot(m, a_ref[...], preferred_element_type=jnp.float32)
        + jnp.dot(x, bmat_ref[...], preferred_element_type=jnp.float32)
        + b1_ref[...], 0.0)
    h = jnp.maximum(
        jnp.dot(h, w2_ref[...], preferred_element_type=jnp.float32)
        + b2_ref[...], 0.0)
    res = (jnp.dot(h, w3_ref[...], preferred_element_type=jnp.float32)
           + b3_ref[...])
    xn = x + jnp.maximum(res, 0.0)
    xo_ref[...] = xn
    x4o_ref[...] = jnp.concatenate(
        [xn[:, 0:3], xn[:, FEAT - 1:FEAT],
         jnp.zeros((xn.shape[0], 4), jnp.float32)], axis=1)


_node_mlp = pl.pallas_call(
    _node_mlp_body,
    grid=(N // BLK_N,),
    in_specs=[
        pl.BlockSpec((NC, BLK_N, 8), lambda i: (0, i, 0)),
        pl.BlockSpec((BLK_N, FEAT), lambda i: (i, 0)),
        pl.BlockSpec((8, HID), lambda i: (0, 0)),
        pl.BlockSpec((FEAT, HID), lambda i: (0, 0)),
        pl.BlockSpec((1, HID), lambda i: (0, 0)),
        pl.BlockSpec((HID, HID), lambda i: (0, 0)),
        pl.BlockSpec((1, HID), lambda i: (0, 0)),
        pl.BlockSpec((HID, FEAT), lambda i: (0, 0)),
        pl.BlockSpec((1, FEAT), lambda i: (0, 0)),
    ],
    out_specs=[
        pl.BlockSpec((BLK_N, FEAT), lambda i: (i, 0)),
        pl.BlockSpec((BLK_N, 8), lambda i: (i, 0)),
    ],
    out_shape=[
        jax.ShapeDtypeStruct((N, FEAT), jnp.float32),
        jax.ShapeDtypeStruct((N, 8), jnp.float32),
    ],
    compiler_params=pltpu.CompilerParams(dimension_semantics=("parallel",)),
)


def _decoder_body(x_ref, w1_ref, b1_ref, w2_ref, b2_ref, w3_ref, b3_ref,
                  w4_ref, b4_ref, o_ref):
    h = jnp.maximum(
        jnp.dot(x_ref[...], w1_ref[...], preferred_element_type=jnp.float32)
        + b1_ref[...], 0.0)
    h = jnp.maximum(
        jnp.dot(h, w2_ref[...], preferred_element_type=jnp.float32)
        + b2_ref[...], 0.0)
    h = jnp.maximum(
        jnp.dot(h, w3_ref[...], preferred_element_type=jnp.float32)
        + b3_ref[...], 0.0)
    o_ref[...] = (jnp.dot(h, w4_ref[...], preferred_element_type=jnp.float32)
                  + b4_ref[...])


_decoder = pl.pallas_call(
    _decoder_body,
    grid=(N // BLK_N,),
    in_specs=[
        pl.BlockSpec((BLK_N, FEAT), lambda i: (i, 0)),
        pl.BlockSpec((FEAT, HID), lambda i: (0, 0)),
        pl.BlockSpec((1, HID), lambda i: (0, 0)),
        pl.BlockSpec((HID, HID), lambda i: (0, 0)),
        pl.BlockSpec((1, HID), lambda i: (0, 0)),
        pl.BlockSpec((HID, HID), lambda i: (0, 0)),
        pl.BlockSpec((1, HID), lambda i: (0, 0)),
        pl.BlockSpec((HID, 8), lambda i: (0, 0)),
        pl.BlockSpec((1, 8), lambda i: (0, 0)),
    ],
    out_specs=pl.BlockSpec((BLK_N, 8), lambda i: (i, 0)),
    out_shape=jax.ShapeDtypeStruct((N, 8), jnp.float32),
    compiler_params=pltpu.CompilerParams(dimension_semantics=("parallel",)),
)


def kernel(x, edge_index, mode, eW1, eb1, eW2, eb2, eW3, eb3, nW1, nb1, nW2,
           nb2, nW3, nb3, dW1, db1, dW2, db2, dW3, db3, dW4, db4):
    del mode
    row = edge_index[0]
    col = edge_index[1]
    col2d = col.reshape(NPAN, PAN)
    zeros_n8 = jnp.zeros((N, 8), jnp.float32)

    eW1p = (jnp.zeros((8, HID), jnp.float32)
            .at[:3].set(eW1[:3]).at[3].set(eW1[4]).at[4].set(eW1[3])
            .at[5].set(eb1)).astype(jnp.bfloat16)
    eW2b = eW2.astype(jnp.bfloat16)
    eb2r = eb2.reshape(1, HID)
    eW3p = (jnp.zeros((HID, 8), jnp.float32).at[:, :5].set(eW3)
            ).astype(jnp.bfloat16)
    eb3p = jnp.zeros((1, 8), jnp.float32).at[0, :5].set(eb3).at[0, 5].set(1.0)

    nA = jnp.zeros((8, HID), jnp.float32).at[:5].set(nW1[:5])
    nB = jnp.zeros((FEAT, HID), jnp.float32).at[3:].set(nW1[5:])
    nb1r = nb1.reshape(1, HID)
    nb2r = nb2.reshape(1, HID)
    nb3r = nb3.reshape(1, FEAT)

    db1r = db1.reshape(1, HID)
    db2r = db2.reshape(1, HID)
    db3r = db3.reshape(1, HID)
    dW4p = jnp.zeros((HID, 8), jnp.float32).at[:, :3].set(dW4)
    db4p = jnp.zeros((1, 8), jnp.float32).at[0, :3].set(db4)

    x4 = jnp.concatenate(
        [x[:, :3], x[:, FEAT - 1:], jnp.zeros((N, 4), jnp.float32)], axis=1)

    gather_call, scatter_call = _sc_calls()
    for _ in range(3):
        epre = gather_call(x4, row, col)
        eattr = _edge_mlp(epre, eW1p, eW2b, eb2r, eW3p, eb3p)
        agg2 = scatter_call(eattr, col2d, zeros_n8)
        x, x4 = _node_mlp(agg2, x, nA, nB, nb1r, nW2, nb2r, nW3, nb3r)

    out8 = _decoder(x, dW1, db1r, dW2, db2r, dW3, db3r, dW4p, db4p)
    return out8[:, :3]


# speedup vs baseline: 11.4716x; 1.1195x over previous
import functools

import jax
import jax.numpy as jnp
from jax import lax
from jax.experimental import pallas as pl
from jax.experimental.pallas import tpu as pltpu
from jax.experimental.pallas import tpu_sc as plsc

N = 10000
E = 320000
FEAT = 128
HID = 256

NC = 2
NS = 16
NW = NC * NS
PAN = 128
NPAN = E // PAN
HPAN = NPAN // 2
PPT = HPAN // NW
PREM = HPAN - PPT * NW
PCH = 13
ZR = N // NS

_SC_PARAMS = pltpu.CompilerParams(needs_layout_passes=False,
                                  use_tc_tiling_on_sc=False)


def _make_gather_body(abs_off):
  def _gather_body(x4_hbm, row_hbm, col_hbm, out_hbm, tab_v, ridx_v, cidx_v,
                   obuf_v):
    cid = lax.axis_index("c")
    sid = lax.axis_index("s")
    wid = cid * NS + sid
    pltpu.sync_copy(x4_hbm, tab_v)

    def do_panels(pbase, npan):
        ne = npan * PAN
        eb = (pbase + abs_off) * PAN
        pltpu.sync_copy(row_hbm.at[pl.ds(eb, ne)], ridx_v.at[pl.ds(0, ne)])
        pltpu.sync_copy(col_hbm.at[pl.ds(eb, ne)], cidx_v.at[pl.ds(0, ne)])

        def panel(jj, carry):
            for g in range(8):
                o = jj * PAN + g * 16
                r = ridx_v[pl.ds(o, 16)]
                cc = cidx_v[pl.ds(o, 16)]
                d = []
                for col in range(4):
                    cv = jnp.full((16,), col, jnp.int32)
                    sv = plsc.load_gather(tab_v, [r, cv])
                    dv = plsc.load_gather(tab_v, [cc, cv])
                    d.append(dv - sv)
                    obuf_v[jj, col, pl.ds(g * 16, 16)] = d[col]
                nsq = d[0] * d[0] + d[1] * d[1] + d[2] * d[2]
                i = plsc.bitcast(nsq, jnp.int32)
                i = 0x5F3759DF - lax.shift_right_logical(i, 1)
                y = plsc.bitcast(i, jnp.float32)
                for _ in range(3):
                    y = y * (1.5 - 0.5 * nsq * y * y)
                nrm = jnp.where(nsq > 0.0, nsq * y, 0.0)
                obuf_v[jj, 4, pl.ds(g * 16, 16)] = nrm
                obuf_v[jj, 5, pl.ds(g * 16, 16)] = jnp.full((16,), 1.0,
                                                            jnp.float32)
            return carry

        lax.fori_loop(0, npan, panel, 0)
        pltpu.sync_copy(obuf_v.at[pl.ds(0, npan)],
                        out_hbm.at[pl.ds(pbase, npan)])

    for ch in range(PPT // PCH):
        do_panels(wid * PPT + ch * PCH, PCH)

    @pl.when(wid < PREM)
    def _():
        do_panels(NW * PPT + wid, 1)

  return _gather_body


def _make_scatter_body(abs_off):
  def _scatter_body(ea_hbm, col2d_hbm, init_hbm, out_hbm, shared, ebuf_v,
                    ibuf_v, rbuf_v):
    cid = lax.axis_index("c")
    sid = lax.axis_index("s")
    wid = cid * NS + sid
    pltpu.sync_copy(init_hbm.at[cid, pl.ds(sid * ZR, ZR)],
                    shared.at[pl.ds(sid * ZR, ZR)])
    zero16 = jnp.zeros((16,), jnp.float32)
    for g in range(8):
        rows = lax.iota(jnp.int32, 16) + g * 16
        plsc.store_scatter(rbuf_v, [rows, jnp.full((16,), 6, jnp.int32)],
                           zero16)
        plsc.store_scatter(rbuf_v, [rows, jnp.full((16,), 7, jnp.int32)],
                           zero16)
    plsc.subcore_barrier()

    def do_panels(pbase, npan):
        pltpu.sync_copy(ea_hbm.at[pl.ds(pbase, npan)],
                        ebuf_v.at[pl.ds(0, npan)])
        pltpu.sync_copy(col2d_hbm.at[pl.ds(pbase + abs_off, npan)],
                        ibuf_v.at[pl.ds(0, npan)])

        def panel(jj, carry):
            for g in range(8):
                rows = lax.iota(jnp.int32, 16) + g * 16
                for c in range(6):
                    v = ebuf_v[jj, c, pl.ds(g * 16, 16)]
                    plsc.store_scatter(rbuf_v, [rows,
                                                jnp.full((16,), c, jnp.int32)],
                                       v)
            pltpu.sync_copy(rbuf_v, shared.at[ibuf_v.at[jj]], add=True)
            return carry

        lax.fori_loop(0, npan, panel, 0)

    for ch in range(PPT // PCH):
        do_panels(wid * PPT + ch * PCH, PCH)

    @pl.when(wid < PREM)
    def _():
        do_panels(NW * PPT + wid, 1)

    plsc.subcore_barrier()
    pltpu.sync_copy(shared.at[pl.ds(sid * ZR, ZR)],
                    out_hbm.at[cid, pl.ds(sid * ZR, ZR)])

  return _scatter_body


@functools.lru_cache(maxsize=None)
def _sc_calls():
    mesh = plsc.VectorSubcoreMesh(core_axis_name="c", subcore_axis_name="s",
                                  num_cores=NC, num_subcores=NS)
    gather_calls = []
    scatter_calls = []
    for half in range(2):
        gather_calls.append(pl.kernel(
            _make_gather_body(half * HPAN),
            out_type=jax.ShapeDtypeStruct((HPAN, 8, PAN), jnp.float32),
            mesh=mesh,
            compiler_params=_SC_PARAMS,
            scratch_types=[
                pltpu.VMEM((N, 8), jnp.float32),
                pltpu.VMEM((PCH * PAN,), jnp.int32),
                pltpu.VMEM((PCH * PAN,), jnp.int32),
                pltpu.VMEM((PCH, 8, PAN), jnp.float32),
            ],
        ))
        scatter_calls.append(pl.kernel(
            _make_scatter_body(half * HPAN),
            out_type=jax.ShapeDtypeStruct((NC, N, 8), jnp.float32),
            mesh=mesh,
            compiler_params=_SC_PARAMS,
            scratch_types=[
                pltpu.VMEM_SHARED((N, 8), jnp.float32),
                pltpu.VMEM((PCH, 8, PAN), jnp.float32),
                pltpu.VMEM((PCH, PAN), jnp.int32),
                pltpu.VMEM((PAN, 8), jnp.float32),
            ],
        ))
    return gather_calls, scatter_calls


BP = 50
BLK_N = 2000


def _edge_mlp_body(e_ref, w1_ref, w2_ref, b2_ref, w3_ref,
                   b3_ref, o_ref):
    e3 = e_ref[...]
    e = jnp.transpose(e3, (0, 2, 1)).reshape(BP * PAN, 8)
    h = jnp.maximum(
        jnp.dot(e.astype(jnp.bfloat16), w1_ref[...],
                preferred_element_type=jnp.float32), 0.0)
    h = jnp.maximum(
        jnp.dot(h.astype(jnp.bfloat16), w2_ref[...],
                preferred_element_type=jnp.float32)
        + b2_ref[...], 0.0)
    o = (jnp.dot(h.astype(jnp.bfloat16), w3_ref[...],
                 preferred_element_type=jnp.float32)
         + b3_ref[...])
    o_ref[...] = jnp.transpose(o.reshape(BP, PAN, 8), (0, 2, 1))


_edge_mlp = pl.pallas_call(
    _edge_mlp_body,
    grid=(HPAN // BP,),
    in_specs=[
        pl.BlockSpec((BP, 8, PAN), lambda i: (i, 0, 0)),
        pl.BlockSpec((8, HID), lambda i: (0, 0)),
        pl.BlockSpec((HID, HID), lambda i: (0, 0)),
        pl.BlockSpec((1, HID), lambda i: (0, 0)),
        pl.BlockSpec((HID, 8), lambda i: (0, 0)),
        pl.BlockSpec((1, 8), lambda i: (0, 0)),
    ],
    out_specs=pl.BlockSpec((BP, 8, PAN), lambda i: (i, 0, 0)),
    out_shape=jax.ShapeDtypeStruct((HPAN, 8, PAN), jnp.float32),
    compiler_params=pltpu.CompilerParams(dimension_semantics=("parallel",)),
)


def _node_mlp_body(agg_ref, x_ref, a_ref, bmat_ref, b1_ref, w2_ref, b2_ref,
                   w3_ref, b3_ref, xo_ref, x4o_ref):
    agg2 = agg_ref[...]
    agg = agg2[0] + agg2[1]
    cnt = agg[:, 5:6]
    inv = 1.0 / jnp.maximum(cnt, 1.0)
    m = agg * inv
    x = x_ref[...]
    h = jnp.maximum(
        jnp.dot(m, a_ref[...], preferred_element_type=jnp.float32)
        + jnp.dot(x, bmat_ref[...], preferred_element_type=jnp.float32)
        + b1_ref[...], 0.0)
    h = jnp.maximum(
        jnp.dot(h, w2_ref[...], preferred_element_type=jnp.float32)
        + b2_ref[...], 0.0)
    res = (jnp.dot(h, w3_ref[...], preferred_element_type=jnp.float32)
           + b3_ref[...])
    xn = x + jnp.maximum(res, 0.0)
    xo_ref[...] = xn
    x4o_ref[...] = jnp.concatenate(
        [xn[:, 0:3], xn[:, FEAT - 1:FEAT],
         jnp.zeros((xn.shape[0], 4), jnp.float32)], axis=1)


_node_mlp = pl.pallas_call(
    _node_mlp_body,
    grid=(N // BLK_N,),
    in_specs=[
        pl.BlockSpec((NC, BLK_N, 8), lambda i: (0, i, 0)),
        pl.BlockSpec((BLK_N, FEAT), lambda i: (i, 0)),
        pl.BlockSpec((8, HID), lambda i: (0, 0)),
        pl.BlockSpec((FEAT, HID), lambda i: (0, 0)),
        pl.BlockSpec((1, HID), lambda i: (0, 0)),
        pl.BlockSpec((HID, HID), lambda i: (0, 0)),
        pl.BlockSpec((1, HID), lambda i: (0, 0)),
        pl.BlockSpec((HID, FEAT), lambda i: (0, 0)),
        pl.BlockSpec((1, FEAT), lambda i: (0, 0)),
    ],
    out_specs=[
        pl.BlockSpec((BLK_N, FEAT), lambda i: (i, 0)),
        pl.BlockSpec((BLK_N, 8), lambda i: (i, 0)),
    ],
    out_shape=[
        jax.ShapeDtypeStruct((N, FEAT), jnp.float32),
        jax.ShapeDtypeStruct((N, 8), jnp.float32),
    ],
    compiler_params=pltpu.CompilerParams(dimension_semantics=("parallel",)),
)


def _decoder_body(x_ref, w1_ref, b1_ref, w2_ref, b2_ref, w3_ref, b3_ref,
                  w4_ref, b4_ref, o_ref):
    h = jnp.maximum(
        jnp.dot(x_ref[...], w1_ref[...], preferred_element_type=jnp.float32)
        + b1_ref[...], 0.0)
    h = jnp.maximum(
        jnp.dot(h, w2_ref[...], preferred_element_type=jnp.float32)
        + b2_ref[...], 0.0)
    h = jnp.maximum(
        jnp.dot(h, w3_ref[...], preferred_element_type=jnp.float32)
        + b3_ref[...], 0.0)
    o_ref[...] = (jnp.dot(h, w4_ref[...], preferred_element_type=jnp.float32)
                  + b4_ref[...])


_decoder = pl.pallas_call(
    _decoder_body,
    grid=(N // BLK_N,),
    in_specs=[
        pl.BlockSpec((BLK_N, FEAT), lambda i: (i, 0)),
        pl.BlockSpec((FEAT, HID), lambda i: (0, 0)),
        pl.BlockSpec((1, HID), lambda i: (0, 0)),
        pl.BlockSpec((HID, HID), lambda i: (0, 0)),
        pl.BlockSpec((1, HID), lambda i: (0, 0)),
        pl.BlockSpec((HID, HID), lambda i: (0, 0)),
        pl.BlockSpec((1, HID), lambda i: (0, 0)),
        pl.BlockSpec((HID, 8), lambda i: (0, 0)),
        pl.BlockSpec((1, 8), lambda i: (0, 0)),
    ],
    out_specs=pl.BlockSpec((BLK_N, 8), lambda i: (i, 0)),
    out_shape=jax.ShapeDtypeStruct((N, 8), jnp.float32),
    compiler_params=pltpu.CompilerParams(dimension_semantics=("parallel",)),
)


def kernel(x, edge_index, mode, eW1, eb1, eW2, eb2, eW3, eb3, nW1, nb1, nW2,
           nb2, nW3, nb3, dW1, db1, dW2, db2, dW3, db3, dW4, db4):
    del mode
    row = edge_index[0]
    col = edge_index[1]
    col2d = col.reshape(NPAN, PAN)
    zeros_init = jnp.zeros((NC, N, 8), jnp.float32)

    eW1p = (jnp.zeros((8, HID), jnp.float32)
            .at[:3].set(eW1[:3]).at[3].set(eW1[4]).at[4].set(eW1[3])
            .at[5].set(eb1)).astype(jnp.bfloat16)
    eW2b = eW2.astype(jnp.bfloat16)
    eb2r = eb2.reshape(1, HID)
    eW3p = (jnp.zeros((HID, 8), jnp.float32).at[:, :5].set(eW3)
            ).astype(jnp.bfloat16)
    eb3p = jnp.zeros((1, 8), jnp.float32).at[0, :5].set(eb3).at[0, 5].set(1.0)

    nA = jnp.zeros((8, HID), jnp.float32).at[:5].set(nW1[:5])
    nB = jnp.zeros((FEAT, HID), jnp.float32).at[3:].set(nW1[5:])
    nb1r = nb1.reshape(1, HID)
    nb2r = nb2.reshape(1, HID)
    nb3r = nb3.reshape(1, FEAT)

    db1r = db1.reshape(1, HID)
    db2r = db2.reshape(1, HID)
    db3r = db3.reshape(1, HID)
    dW4p = jnp.zeros((HID, 8), jnp.float32).at[:, :3].set(dW4)
    db4p = jnp.zeros((1, 8), jnp.float32).at[0, :3].set(db4)

    x4 = jnp.concatenate(
        [x[:, :3], x[:, FEAT - 1:], jnp.zeros((N, 4), jnp.float32)], axis=1)

    gather_calls, scatter_calls = _sc_calls()
    for _ in range(3):
        ep0 = gather_calls[0](x4, row, col)
        ep1 = gather_calls[1](x4, row, col)
        ea0 = _edge_mlp(ep0, eW1p, eW2b, eb2r, eW3p, eb3p)
        ea1 = _edge_mlp(ep1, eW1p, eW2b, eb2r, eW3p, eb3p)
        agg_h0 = scatter_calls[0](ea0, col2d, zeros_init)
        agg2 = scatter_calls[1](ea1, col2d, agg_h0)
        x, x4 = _node_mlp(agg2, x, nA, nB, nb1r, nW2, nb2r, nW3, nb3r)

    out8 = _decoder(x, dW1, db1r, dW2, db2r, dW3, db3r, dW4p, db4p)
    return out8[:, :3]


# speedup vs baseline: 11.7797x; 1.0269x over previous
import functools

import jax
import jax.numpy as jnp
from jax import lax
from jax.experimental import pallas as pl
from jax.experimental.pallas import tpu as pltpu
from jax.experimental.pallas import tpu_sc as plsc

N = 10000
E = 320000
FEAT = 128
HID = 256

NC = 2
NS = 16
NW = NC * NS
PAN = 128
NPAN = E // PAN
HPAN = NPAN // 2
PPT = HPAN // NW
PREM = HPAN - PPT * NW
PCH = 13
ZR = N // NS

_SC_PARAMS = pltpu.CompilerParams(needs_layout_passes=False,
                                  use_tc_tiling_on_sc=False)


def _make_gather_body(abs_off):
  NCH = PPT // PCH

  def _gather_body(x4_hbm, row_hbm, col_hbm, out_hbm, tab_v, ridx_v, cidx_v,
                   obuf_v, isem, osem):
    cid = lax.axis_index("c")
    sid = lax.axis_index("s")
    wid = cid * NS + sid
    pltpu.sync_copy(x4_hbm, tab_v)

    def start_in(ch, buf):
        eb = (wid * PPT + ch * PCH + abs_off) * PAN
        ne = PCH * PAN
        pltpu.async_copy(row_hbm.at[pl.ds(eb, ne)],
                         ridx_v.at[buf], isem.at[buf])
        pltpu.async_copy(col_hbm.at[pl.ds(eb, ne)],
                         cidx_v.at[buf], isem.at[buf])

    def wait_in(ch, buf):
        eb = (wid * PPT + ch * PCH + abs_off) * PAN
        ne = PCH * PAN
        pltpu.make_async_copy(row_hbm.at[pl.ds(eb, ne)],
                              ridx_v.at[buf], isem.at[buf]).wait()
        pltpu.make_async_copy(col_hbm.at[pl.ds(eb, ne)],
                              cidx_v.at[buf], isem.at[buf]).wait()

    def compute(npan, buf):
        def panel(jj, carry):
            for g in range(8):
                o = jj * PAN + g * 16
                r = ridx_v[buf, pl.ds(o, 16)]
                cc = cidx_v[buf, pl.ds(o, 16)]
                d = []
                for col in range(4):
                    cv = jnp.full((16,), col, jnp.int32)
                    sv = plsc.load_gather(tab_v, [r, cv])
                    dv = plsc.load_gather(tab_v, [cc, cv])
                    d.append(dv - sv)
                    obuf_v[buf, jj, col, pl.ds(g * 16, 16)] = d[col]
                nsq = d[0] * d[0] + d[1] * d[1] + d[2] * d[2]
                i = plsc.bitcast(nsq, jnp.int32)
                i = 0x5F3759DF - lax.shift_right_logical(i, 1)
                y = plsc.bitcast(i, jnp.float32)
                for _ in range(3):
                    y = y * (1.5 - 0.5 * nsq * y * y)
                nrm = jnp.where(nsq > 0.0, nsq * y, 0.0)
                obuf_v[buf, jj, 4, pl.ds(g * 16, 16)] = nrm
                obuf_v[buf, jj, 5, pl.ds(g * 16, 16)] = jnp.full(
                    (16,), 1.0, jnp.float32)
            return carry

        lax.fori_loop(0, npan, panel, 0)

    def start_out(ch, buf):
        pltpu.async_copy(obuf_v.at[buf],
                         out_hbm.at[pl.ds(wid * PPT + ch * PCH, PCH)],
                         osem.at[buf])

    def wait_out(ch, buf):
        pltpu.make_async_copy(obuf_v.at[buf],
                              out_hbm.at[pl.ds(wid * PPT + ch * PCH, PCH)],
                              osem.at[buf]).wait()

    start_in(0, 0)
    for ch in range(NCH):
        buf = ch % 2
        if ch + 1 < NCH:
            start_in(ch + 1, 1 - buf)
        wait_in(ch, buf)
        if ch >= 2:
            wait_out(ch - 2, buf)
        compute(PCH, buf)
        start_out(ch, buf)
    for ch in range(max(NCH - 2, 0), NCH):
        wait_out(ch, ch % 2)

    @pl.when(wid < PREM)
    def _():
        pbase = NW * PPT + wid
        eb = (pbase + abs_off) * PAN
        pltpu.sync_copy(row_hbm.at[pl.ds(eb, PAN)],
                        ridx_v.at[0, pl.ds(0, PAN)])
        pltpu.sync_copy(col_hbm.at[pl.ds(eb, PAN)],
                        cidx_v.at[0, pl.ds(0, PAN)])
        compute(1, 0)
        pltpu.sync_copy(obuf_v.at[0, pl.ds(0, 1)],
                        out_hbm.at[pl.ds(pbase, 1)])

  return _gather_body


def _make_scatter_body(abs_off):
  NCH = PPT // PCH

  def _scatter_body(ea_hbm, col2d_hbm, init_hbm, out_hbm, shared, ebuf_v,
                    ibuf_v, rbuf_v, isem, ssem):
    cid = lax.axis_index("c")
    sid = lax.axis_index("s")
    wid = cid * NS + sid
    pltpu.sync_copy(init_hbm.at[cid, pl.ds(sid * ZR, ZR)],
                    shared.at[pl.ds(sid * ZR, ZR)])
    zero16 = jnp.zeros((16,), jnp.float32)
    for rb in range(2):
        for g in range(8):
            rows = lax.iota(jnp.int32, 16) + g * 16
            plsc.store_scatter(rbuf_v.at[rb],
                               [rows, jnp.full((16,), 6, jnp.int32)], zero16)
            plsc.store_scatter(rbuf_v.at[rb],
                               [rows, jnp.full((16,), 7, jnp.int32)], zero16)
    plsc.subcore_barrier()

    def start_in(ch, buf):
        pb = wid * PPT + ch * PCH
        pltpu.async_copy(ea_hbm.at[pl.ds(pb, PCH)], ebuf_v.at[buf],
                         isem.at[buf])
        pltpu.async_copy(col2d_hbm.at[pl.ds(pb + abs_off, PCH)],
                         ibuf_v.at[buf], isem.at[buf])

    def wait_in(ch, buf):
        pb = wid * PPT + ch * PCH
        pltpu.make_async_copy(ea_hbm.at[pl.ds(pb, PCH)], ebuf_v.at[buf],
                              isem.at[buf]).wait()
        pltpu.make_async_copy(col2d_hbm.at[pl.ds(pb + abs_off, PCH)],
                              ibuf_v.at[buf], isem.at[buf]).wait()

    def build(buf, jj, rb):
        for g in range(8):
            rows = lax.iota(jnp.int32, 16) + g * 16
            for c in range(6):
                v = ebuf_v[buf, jj, c, pl.ds(g * 16, 16)]
                plsc.store_scatter(rbuf_v.at[rb],
                                   [rows, jnp.full((16,), c, jnp.int32)], v)

    def drain(rb):
        pltpu.make_async_copy(rbuf_v.at[rb], shared.at[pl.ds(0, PAN)],
                              ssem.at[rb]).wait()

    start_in(0, 0)
    for ch in range(NCH):
        buf = ch % 2
        if ch + 1 < NCH:
            start_in(ch + 1, 1 - buf)
        wait_in(ch, buf)
        for jj in range(PCH):
            rb = jj % 2
            if jj >= 2:
                drain(rb)
            build(buf, jj, rb)
            pltpu.async_copy(rbuf_v.at[rb], shared.at[ibuf_v.at[buf, jj]],
                             ssem.at[rb], add=True)
        drain(0)
        drain(1)

    @pl.when(wid < PREM)
    def _():
        pbase = NW * PPT + wid
        pltpu.sync_copy(ea_hbm.at[pl.ds(pbase, 1)], ebuf_v.at[0, pl.ds(0, 1)])
        pltpu.sync_copy(col2d_hbm.at[pl.ds(pbase + abs_off, 1)],
                        ibuf_v.at[0, pl.ds(0, 1)])
        build(0, 0, 0)
        pltpu.sync_copy(rbuf_v.at[0], shared.at[ibuf_v.at[0, 0]], add=True)

    plsc.subcore_barrier()
    pltpu.sync_copy(shared.at[pl.ds(sid * ZR, ZR)],
                    out_hbm.at[cid, pl.ds(sid * ZR, ZR)])

  return _scatter_body


@functools.lru_cache(maxsize=None)
def _sc_calls():
    mesh = plsc.VectorSubcoreMesh(core_axis_name="c", subcore_axis_name="s",
                                  num_cores=NC, num_subcores=NS)
    gather_calls = []
    scatter_calls = []
    for half in range(2):
        gather_calls.append(pl.kernel(
            _make_gather_body(half * HPAN),
            out_type=jax.ShapeDtypeStruct((HPAN, 8, PAN), jnp.float32),
            mesh=mesh,
            compiler_params=_SC_PARAMS,
            scratch_types=[
                pltpu.VMEM((N, 8), jnp.float32),
                pltpu.VMEM((2, PCH * PAN), jnp.int32),
                pltpu.VMEM((2, PCH * PAN), jnp.int32),
                pltpu.VMEM((2, PCH, 8, PAN), jnp.float32),
                pltpu.SemaphoreType.DMA((2,)),
                pltpu.SemaphoreType.DMA((2,)),
            ],
        ))
        scatter_calls.append(pl.kernel(
            _make_scatter_body(half * HPAN),
            out_type=jax.ShapeDtypeStruct((NC, N, 8), jnp.float32),
            mesh=mesh,
            compiler_params=_SC_PARAMS,
            scratch_types=[
                pltpu.VMEM_SHARED((N, 8), jnp.float32),
                pltpu.VMEM((2, PCH, 8, PAN), jnp.float32),
                pltpu.VMEM((2, PCH, PAN), jnp.int32),
                pltpu.VMEM((2, PAN, 8), jnp.float32),
                pltpu.SemaphoreType.DMA((2,)),
                pltpu.SemaphoreType.DMA((2,)),
            ],
        ))
    return gather_calls, scatter_calls


BP = 50
BLK_N = 2000


def _edge_mlp_body(e_ref, w1_ref, w2_ref, b2_ref, w3_ref,
                   b3_ref, o_ref):
    e3 = e_ref[...]
    e = jnp.transpose(e3, (0, 2, 1)).reshape(BP * PAN, 8)
    h = jnp.maximum(
        jnp.dot(e.astype(jnp.bfloat16), w1_ref[...],
                preferred_element_type=jnp.float32), 0.0)
    h = jnp.maximum(
        jnp.dot(h.astype(jnp.bfloat16), w2_ref[...],
                preferred_element_type=jnp.float32)
        + b2_ref[...], 0.0)
    o = (jnp.dot(h.astype(jnp.bfloat16), w3_ref[...],
                 preferred_element_type=jnp.float32)
         + b3_ref[...])
    o_ref[...] = jnp.transpose(o.reshape(BP, PAN, 8), (0, 2, 1))


_edge_mlp = pl.pallas_call(
    _edge_mlp_body,
    grid=(HPAN // BP,),
    in_specs=[
        pl.BlockSpec((BP, 8, PAN), lambda i: (i, 0, 0)),
        pl.BlockSpec((8, HID), lambda i: (0, 0)),
        pl.BlockSpec((HID, HID), lambda i: (0, 0)),
        pl.BlockSpec((1, HID), lambda i: (0, 0)),
        pl.BlockSpec((HID, 8), lambda i: (0, 0)),
        pl.BlockSpec((1, 8), lambda i: (0, 0)),
    ],
    out_specs=pl.BlockSpec((BP, 8, PAN), lambda i: (i, 0, 0)),
    out_shape=jax.ShapeDtypeStruct((HPAN, 8, PAN), jnp.float32),
    compiler_params=pltpu.CompilerParams(dimension_semantics=("parallel",)),
)


def _node_mlp_body(agg_ref, x_ref, a_ref, bmat_ref, b1_ref, w2_ref, b2_ref,
                   w3_ref, b3_ref, xo_ref, x4o_ref):
    agg2 = agg_ref[...]
    agg = agg2[0] + agg2[1]
    cnt = agg[:, 5:6]
    inv = 1.0 / jnp.maximum(cnt, 1.0)
    m = agg * inv
    x = x_ref[...]
    h = jnp.maximum(
        jnp.dot(m, a_ref[...], preferred_element_type=jnp.float32)
        + jnp.dot(x, bmat_ref[...], preferred_element_type=jnp.float32)
        + b1_ref[...], 0.0)
    h = jnp.maximum(
        jnp.dot(h, w2_ref[...], preferred_element_type=jnp.float32)
        + b2_ref[...], 0.0)
    res = (jnp.dot(h, w3_ref[...], preferred_element_type=jnp.float32)
           + b3_ref[...])
    xn = x + jnp.maximum(res, 0.0)
    xo_ref[...] = xn
    x4o_ref[...] = jnp.concatenate(
        [xn[:, 0:3], xn[:, FEAT - 1:FEAT],
         jnp.zeros((xn.shape[0], 4), jnp.float32)], axis=1)


_node_mlp = pl.pallas_call(
    _node_mlp_body,
    grid=(N // BLK_N,),
    in_specs=[
        pl.BlockSpec((NC, BLK_N, 8), lambda i: (0, i, 0)),
        pl.BlockSpec((BLK_N, FEAT), lambda i: (i, 0)),
        pl.BlockSpec((8, HID), lambda i: (0, 0)),
        pl.BlockSpec((FEAT, HID), lambda i: (0, 0)),
        pl.BlockSpec((1, HID), lambda i: (0, 0)),
        pl.BlockSpec((HID, HID), lambda i: (0, 0)),
        pl.BlockSpec((1, HID), lambda i: (0, 0)),
        pl.BlockSpec((HID, FEAT), lambda i: (0, 0)),
        pl.BlockSpec((1, FEAT), lambda i: (0, 0)),
    ],
    out_specs=[
        pl.BlockSpec((BLK_N, FEAT), lambda i: (i, 0)),
        pl.BlockSpec((BLK_N, 8), lambda i: (i, 0)),
    ],
    out_shape=[
        jax.ShapeDtypeStruct((N, FEAT), jnp.float32),
        jax.ShapeDtypeStruct((N, 8), jnp.float32),
    ],
    compiler_params=pltpu.CompilerParams(dimension_semantics=("parallel",)),
)


def _decoder_body(x_ref, w1_ref, b1_ref, w2_ref, b2_ref, w3_ref, b3_ref,
                  w4_ref, b4_ref, o_ref):
    h = jnp.maximum(
        jnp.dot(x_ref[...], w1_ref[...], preferred_element_type=jnp.float32)
        + b1_ref[...], 0.0)
    h = jnp.maximum(
        jnp.dot(h, w2_ref[...], preferred_element_type=jnp.float32)
        + b2_ref[...], 0.0)
    h = jnp.maximum(
        jnp.dot(h, w3_ref[...], preferred_element_type=jnp.float32)
        + b3_ref[...], 0.0)
    o_ref[...] = (jnp.dot(h, w4_ref[...], preferred_element_type=jnp.float32)
                  + b4_ref[...])


_decoder = pl.pallas_call(
    _decoder_body,
    grid=(N // BLK_N,),
    in_specs=[
        pl.BlockSpec((BLK_N, FEAT), lambda i: (i, 0)),
        pl.BlockSpec((FEAT, HID), lambda i: (0, 0)),
        pl.BlockSpec((1, HID), lambda i: (0, 0)),
        pl.BlockSpec((HID, HID), lambda i: (0, 0)),
        pl.BlockSpec((1, HID), lambda i: (0, 0)),
        pl.BlockSpec((HID, HID), lambda i: (0, 0)),
        pl.BlockSpec((1, HID), lambda i: (0, 0)),
        pl.BlockSpec((HID, 8), lambda i: (0, 0)),
        pl.BlockSpec((1, 8), lambda i: (0, 0)),
    ],
    out_specs=pl.BlockSpec((BLK_N, 8), lambda i: (i, 0)),
    out_shape=jax.ShapeDtypeStruct((N, 8), jnp.float32),
    compiler_params=pltpu.CompilerParams(dimension_semantics=("parallel",)),
)


def kernel(x, edge_index, mode, eW1, eb1, eW2, eb2, eW3, eb3, nW1, nb1, nW2,
           nb2, nW3, nb3, dW1, db1, dW2, db2, dW3, db3, dW4, db4):
    del mode
    row = edge_index[0]
    col = edge_index[1]
    col2d = col.reshape(NPAN, PAN)
    zeros_init = jnp.zeros((NC, N, 8), jnp.float32)

    eW1p = (jnp.zeros((8, HID), jnp.float32)
            .at[:3].set(eW1[:3]).at[3].set(eW1[4]).at[4].set(eW1[3])
            .at[5].set(eb1)).astype(jnp.bfloat16)
    eW2b = eW2.astype(jnp.bfloat16)
    eb2r = eb2.reshape(1, HID)
    eW3p = (jnp.zeros((HID, 8), jnp.float32).at[:, :5].set(eW3)
            ).astype(jnp.bfloat16)
    eb3p = jnp.zeros((1, 8), jnp.float32).at[0, :5].set(eb3).at[0, 5].set(1.0)

    nA = jnp.zeros((8, HID), jnp.float32).at[:5].set(nW1[:5])
    nB = jnp.zeros((FEAT, HID), jnp.float32).at[3:].set(nW1[5:])
    nb1r = nb1.reshape(1, HID)
    nb2r = nb2.reshape(1, HID)
    nb3r = nb3.reshape(1, FEAT)

    db1r = db1.reshape(1, HID)
    db2r = db2.reshape(1, HID)
    db3r = db3.reshape(1, HID)
    dW4p = jnp.zeros((HID, 8), jnp.float32).at[:, :3].set(dW4)
    db4p = jnp.zeros((1, 8), jnp.float32).at[0, :3].set(db4)

    x4 = jnp.concatenate(
        [x[:, :3], x[:, FEAT - 1:], jnp.zeros((N, 4), jnp.float32)], axis=1)

    gather_calls, scatter_calls = _sc_calls()
    for _ in range(3):
        ep0 = gather_calls[0](x4, row, col)
        ep1 = gather_calls[1](x4, row, col)
        ea0 = _edge_mlp(ep0, eW1p, eW2b, eb2r, eW3p, eb3p)
        ea1 = _edge_mlp(ep1, eW1p, eW2b, eb2r, eW3p, eb3p)
        agg_h0 = scatter_calls[0](ea0, col2d, zeros_init)
        agg2 = scatter_calls[1](ea1, col2d, agg_h0)
        x, x4 = _node_mlp(agg2, x, nA, nB, nb1r, nW2, nb2r, nW3, nb3r)

    out8 = _decoder(x, dW1, db1r, dW2, db2r, dW3, db3r, dW4p, db4p)
    return out8[:, :3]


# speedup vs baseline: 12.0108x; 1.0196x over previous
import functools

import jax
import jax.numpy as jnp
from jax import lax
from jax.experimental import pallas as pl
from jax.experimental.pallas import tpu as pltpu
from jax.experimental.pallas import tpu_sc as plsc

N = 10000
E = 320000
FEAT = 128
HID = 256

NC = 2
NS = 16
NW = NC * NS
PAN = 128
NPAN = E // PAN
HPAN = NPAN // 2
PPT = HPAN // NW
PREM = HPAN - PPT * NW
PCH = 13
ZR = N // NS

_SC_PARAMS = pltpu.CompilerParams(needs_layout_passes=False,
                                  use_tc_tiling_on_sc=False)


def _make_gather_body(abs_off):
  NCH = PPT // PCH

  def _gather_body(x4_hbm, row_hbm, col_hbm, out_hbm, tab_v, ridx_v, cidx_v,
                   obuf_v, isem, osem):
    cid = lax.axis_index("c")
    sid = lax.axis_index("s")
    wid = cid * NS + sid
    pltpu.sync_copy(x4_hbm, tab_v)

    def start_in(ch, buf):
        eb = (wid * PPT + ch * PCH + abs_off) * PAN
        ne = PCH * PAN
        pltpu.async_copy(row_hbm.at[pl.ds(eb, ne)],
                         ridx_v.at[buf], isem.at[buf])
        pltpu.async_copy(col_hbm.at[pl.ds(eb, ne)],
                         cidx_v.at[buf], isem.at[buf])

    def wait_in(ch, buf):
        eb = (wid * PPT + ch * PCH + abs_off) * PAN
        ne = PCH * PAN
        pltpu.make_async_copy(row_hbm.at[pl.ds(eb, ne)],
                              ridx_v.at[buf], isem.at[buf]).wait()
        pltpu.make_async_copy(col_hbm.at[pl.ds(eb, ne)],
                              cidx_v.at[buf], isem.at[buf]).wait()

    def compute(npan, buf):
        @plsc.parallel_loop(0, npan, unroll=2)
        def panel(jj):
            for g in range(8):
                o = jj * PAN + g * 16
                r = ridx_v[buf, pl.ds(o, 16)]
                cc = cidx_v[buf, pl.ds(o, 16)]
                d = []
                for col in range(4):
                    cv = jnp.full((16,), col, jnp.int32)
                    sv = plsc.load_gather(tab_v, [r, cv])
                    dv = plsc.load_gather(tab_v, [cc, cv])
                    d.append(dv - sv)
                    obuf_v[buf, jj, col, pl.ds(g * 16, 16)] = d[col]
                nsq = d[0] * d[0] + d[1] * d[1] + d[2] * d[2]
                i = plsc.bitcast(nsq, jnp.int32)
                i = 0x5F3759DF - lax.shift_right_logical(i, 1)
                y = plsc.bitcast(i, jnp.float32)
                for _ in range(3):
                    y = y * (1.5 - 0.5 * nsq * y * y)
                nrm = jnp.where(nsq > 0.0, nsq * y, 0.0)
                obuf_v[buf, jj, 4, pl.ds(g * 16, 16)] = nrm
                obuf_v[buf, jj, 5, pl.ds(g * 16, 16)] = jnp.full(
                    (16,), 1.0, jnp.float32)

    def start_out(ch, buf):
        pltpu.async_copy(obuf_v.at[buf],
                         out_hbm.at[pl.ds(wid * PPT + ch * PCH, PCH)],
                         osem.at[buf])

    def wait_out(ch, buf):
        pltpu.make_async_copy(obuf_v.at[buf],
                              out_hbm.at[pl.ds(wid * PPT + ch * PCH, PCH)],
                              osem.at[buf]).wait()

    start_in(0, 0)
    for ch in range(NCH):
        buf = ch % 2
        if ch + 1 < NCH:
            start_in(ch + 1, 1 - buf)
        wait_in(ch, buf)
        if ch >= 2:
            wait_out(ch - 2, buf)
        compute(PCH, buf)
        start_out(ch, buf)
    for ch in range(max(NCH - 2, 0), NCH):
        wait_out(ch, ch % 2)

    @pl.when(wid < PREM)
    def _():
        pbase = NW * PPT + wid
        eb = (pbase + abs_off) * PAN
        pltpu.sync_copy(row_hbm.at[pl.ds(eb, PAN)],
                        ridx_v.at[0, pl.ds(0, PAN)])
        pltpu.sync_copy(col_hbm.at[pl.ds(eb, PAN)],
                        cidx_v.at[0, pl.ds(0, PAN)])
        compute(1, 0)
        pltpu.sync_copy(obuf_v.at[0, pl.ds(0, 1)],
                        out_hbm.at[pl.ds(pbase, 1)])

  return _gather_body


def _make_scatter_body(abs_off):
  NCH = PPT // PCH

  def _scatter_body(ea_hbm, col2d_hbm, init_hbm, out_hbm, shared, ebuf_v,
                    ibuf_v, rbuf_v, isem, ssem):
    cid = lax.axis_index("c")
    sid = lax.axis_index("s")
    wid = cid * NS + sid
    pltpu.sync_copy(init_hbm.at[cid, pl.ds(sid * ZR, ZR)],
                    shared.at[pl.ds(sid * ZR, ZR)])
    zero16 = jnp.zeros((16,), jnp.float32)
    for rb in range(2):
        for g in range(8):
            rows = lax.iota(jnp.int32, 16) + g * 16
            plsc.store_scatter(rbuf_v.at[rb],
                               [rows, jnp.full((16,), 6, jnp.int32)], zero16)
            plsc.store_scatter(rbuf_v.at[rb],
                               [rows, jnp.full((16,), 7, jnp.int32)], zero16)
    plsc.subcore_barrier()

    def start_in(ch, buf):
        pb = wid * PPT + ch * PCH
        pltpu.async_copy(ea_hbm.at[pl.ds(pb, PCH)], ebuf_v.at[buf],
                         isem.at[buf])
        pltpu.async_copy(col2d_hbm.at[pl.ds(pb + abs_off, PCH)],
                         ibuf_v.at[buf], isem.at[buf])

    def wait_in(ch, buf):
        pb = wid * PPT + ch * PCH
        pltpu.make_async_copy(ea_hbm.at[pl.ds(pb, PCH)], ebuf_v.at[buf],
                              isem.at[buf]).wait()
        pltpu.make_async_copy(col2d_hbm.at[pl.ds(pb + abs_off, PCH)],
                              ibuf_v.at[buf], isem.at[buf]).wait()

    def build(buf, jj, rb):
        for g in range(8):
            rows = lax.iota(jnp.int32, 16) + g * 16
            for c in range(6):
                v = ebuf_v[buf, jj, c, pl.ds(g * 16, 16)]
                plsc.store_scatter(rbuf_v.at[rb],
                                   [rows, jnp.full((16,), c, jnp.int32)], v)

    def drain(rb):
        pltpu.make_async_copy(rbuf_v.at[rb], shared.at[pl.ds(0, PAN)],
                              ssem.at[rb]).wait()

    start_in(0, 0)
    for ch in range(NCH):
        buf = ch % 2
        if ch + 1 < NCH:
            start_in(ch + 1, 1 - buf)
        wait_in(ch, buf)
        for jj in range(PCH):
            rb = jj % 2
            if jj >= 2:
                drain(rb)
            build(buf, jj, rb)
            pltpu.async_copy(rbuf_v.at[rb], shared.at[ibuf_v.at[buf, jj]],
                             ssem.at[rb], add=True)
        drain(0)
        drain(1)

    @pl.when(wid < PREM)
    def _():
        pbase = NW * PPT + wid
        pltpu.sync_copy(ea_hbm.at[pl.ds(pbase, 1)], ebuf_v.at[0, pl.ds(0, 1)])
        pltpu.sync_copy(col2d_hbm.at[pl.ds(pbase + abs_off, 1)],
                        ibuf_v.at[0, pl.ds(0, 1)])
        build(0, 0, 0)
        pltpu.sync_copy(rbuf_v.at[0], shared.at[ibuf_v.at[0, 0]], add=True)

    plsc.subcore_barrier()
    pltpu.sync_copy(shared.at[pl.ds(sid * ZR, ZR)],
                    out_hbm.at[cid, pl.ds(sid * ZR, ZR)])

  return _scatter_body


@functools.lru_cache(maxsize=None)
def _sc_calls():
    mesh = plsc.VectorSubcoreMesh(core_axis_name="c", subcore_axis_name="s",
                                  num_cores=NC, num_subcores=NS)
    gather_calls = []
    scatter_calls = []
    for half in range(2):
        gather_calls.append(pl.kernel(
            _make_gather_body(half * HPAN),
            out_type=jax.ShapeDtypeStruct((HPAN, 8, PAN), jnp.float32),
            mesh=mesh,
            compiler_params=_SC_PARAMS,
            scratch_types=[
                pltpu.VMEM((N, 8), jnp.float32),
                pltpu.VMEM((2, PCH * PAN), jnp.int32),
                pltpu.VMEM((2, PCH * PAN), jnp.int32),
                pltpu.VMEM((2, PCH, 8, PAN), jnp.float32),
                pltpu.SemaphoreType.DMA((2,)),
                pltpu.SemaphoreType.DMA((2,)),
            ],
        ))
        scatter_calls.append(pl.kernel(
            _make_scatter_body(half * HPAN),
            out_type=jax.ShapeDtypeStruct((NC, N, 8), jnp.float32),
            mesh=mesh,
            compiler_params=_SC_PARAMS,
            scratch_types=[
                pltpu.VMEM_SHARED((N, 8), jnp.float32),
                pltpu.VMEM((2, PCH, 8, PAN), jnp.float32),
                pltpu.VMEM((2, PCH, PAN), jnp.int32),
                pltpu.VMEM((2, PAN, 8), jnp.float32),
                pltpu.SemaphoreType.DMA((2,)),
                pltpu.SemaphoreType.DMA((2,)),
            ],
        ))
    return gather_calls, scatter_calls


BP = 50
BLK_N = 2000


def _edge_mlp_body(e_ref, w1_ref, w2_ref, b2_ref, w3_ref,
                   b3_ref, o_ref):
    e3 = e_ref[...]
    e = jnp.transpose(e3, (0, 2, 1)).reshape(BP * PAN, 8)
    h = jnp.maximum(
        jnp.dot(e.astype(jnp.bfloat16), w1_ref[...],
                preferred_element_type=jnp.float32), 0.0)
    h = jnp.maximum(
        jnp.dot(h.astype(jnp.bfloat16), w2_ref[...],
                preferred_element_type=jnp.float32)
        + b2_ref[...], 0.0)
    o = (jnp.dot(h.astype(jnp.bfloat16), w3_ref[...],
                 preferred_element_type=jnp.float32)
         + b3_ref[...])
    o_ref[...] = jnp.transpose(o.reshape(BP, PAN, 8), (0, 2, 1))


_edge_mlp = pl.pallas_call(
    _edge_mlp_body,
    grid=(HPAN // BP,),
    in_specs=[
        pl.BlockSpec((BP, 8, PAN), lambda i: (i, 0, 0)),
        pl.BlockSpec((8, HID), lambda i: (0, 0)),
        pl.BlockSpec((HID, HID), lambda i: (0, 0)),
        pl.BlockSpec((1, HID), lambda i: (0, 0)),
        pl.BlockSpec((HID, 8), lambda i: (0, 0)),
        pl.BlockSpec((1, 8), lambda i: (0, 0)),
    ],
    out_specs=pl.BlockSpec((BP, 8, PAN), lambda i: (i, 0, 0)),
    out_shape=jax.ShapeDtypeStruct((HPAN, 8, PAN), jnp.float32),
    compiler_params=pltpu.CompilerParams(dimension_semantics=("parallel",)),
)


def _node_mlp_body(agg_ref, x_ref, a_ref, bmat_ref, b1_ref, w2_ref, b2_ref,
                   w3_ref, b3_ref, xo_ref, x4o_ref):
    agg2 = agg_ref[...]
    agg = agg2[0] + agg2[1]
    cnt = agg[:, 5:6]
    inv = 1.0 / jnp.maximum(cnt, 1.0)
    m = agg * inv
    x = x_ref[...]
    h = jnp.maximum(
        jnp.dot(m, a_ref[...], preferred_element_type=jnp.float32)
        + jnp.dot(x, bmat_ref[...], preferred_element_type=jnp.float32)
        + b1_ref[...], 0.0)
    h = jnp.maximum(
        jnp.dot(h, w2_ref[...], preferred_element_type=jnp.float32)
        + b2_ref[...], 0.0)
    res = (jnp.dot(h, w3_ref[...], preferred_element_type=jnp.float32)
           + b3_ref[...])
    xn = x + jnp.maximum(res, 0.0)
    xo_ref[...] = xn
    x4o_ref[...] = jnp.concatenate(
        [xn[:, 0:3], xn[:, FEAT - 1:FEAT],
         jnp.zeros((xn.shape[0], 4), jnp.float32)], axis=1)


_node_mlp = pl.pallas_call(
    _node_mlp_body,
    grid=(N // BLK_N,),
    in_specs=[
        pl.BlockSpec((NC, BLK_N, 8), lambda i: (0, i, 0)),
        pl.BlockSpec((BLK_N, FEAT), lambda i: (i, 0)),
        pl.BlockSpec((8, HID), lambda i: (0, 0)),
        pl.BlockSpec((FEAT, HID), lambda i: (0, 0)),
        pl.BlockSpec((1, HID), lambda i: (0, 0)),
        pl.BlockSpec((HID, HID), lambda i: (0, 0)),
        pl.BlockSpec((1, HID), lambda i: (0, 0)),
        pl.BlockSpec((HID, FEAT), lambda i: (0, 0)),
        pl.BlockSpec((1, FEAT), lambda i: (0, 0)),
    ],
    out_specs=[
        pl.BlockSpec((BLK_N, FEAT), lambda i: (i, 0)),
        pl.BlockSpec((BLK_N, 8), lambda i: (i, 0)),
    ],
    out_shape=[
        jax.ShapeDtypeStruct((N, FEAT), jnp.float32),
        jax.ShapeDtypeStruct((N, 8), jnp.float32),
    ],
    compiler_params=pltpu.CompilerParams(dimension_semantics=("parallel",)),
)


def _decoder_body(x_ref, w1_ref, b1_ref, w2_ref, b2_ref, w3_ref, b3_ref,
                  w4_ref, b4_ref, o_ref):
    h = jnp.maximum(
        jnp.dot(x_ref[...], w1_ref[...], preferred_element_type=jnp.float32)
        + b1_ref[...], 0.0)
    h = jnp.maximum(
        jnp.dot(h, w2_ref[...], preferred_element_type=jnp.float32)
        + b2_ref[...], 0.0)
    h = jnp.maximum(
        jnp.dot(h, w3_ref[...], preferred_element_type=jnp.float32)
        + b3_ref[...], 0.0)
    o_ref[...] = (jnp.dot(h, w4_ref[...], preferred_element_type=jnp.float32)
                  + b4_ref[...])


_decoder = pl.pallas_call(
    _decoder_body,
    grid=(N // BLK_N,),
    in_specs=[
        pl.BlockSpec((BLK_N, FEAT), lambda i: (i, 0)),
        pl.BlockSpec((FEAT, HID), lambda i: (0, 0)),
        pl.BlockSpec((1, HID), lambda i: (0, 0)),
        pl.BlockSpec((HID, HID), lambda i: (0, 0)),
        pl.BlockSpec((1, HID), lambda i: (0, 0)),
        pl.BlockSpec((HID, HID), lambda i: (0, 0)),
        pl.BlockSpec((1, HID), lambda i: (0, 0)),
        pl.BlockSpec((HID, 8), lambda i: (0, 0)),
        pl.BlockSpec((1, 8), lambda i: (0, 0)),
    ],
    out_specs=pl.BlockSpec((BLK_N, 8), lambda i: (i, 0)),
    out_shape=jax.ShapeDtypeStruct((N, 8), jnp.float32),
    compiler_params=pltpu.CompilerParams(dimension_semantics=("parallel",)),
)


def kernel(x, edge_index, mode, eW1, eb1, eW2, eb2, eW3, eb3, nW1, nb1, nW2,
           nb2, nW3, nb3, dW1, db1, dW2, db2, dW3, db3, dW4, db4):
    del mode
    row = edge_index[0]
    col = edge_index[1]
    col2d = col.reshape(NPAN, PAN)
    zeros_init = jnp.zeros((NC, N, 8), jnp.float32)

    eW1p = (jnp.zeros((8, HID), jnp.float32)
            .at[:3].set(eW1[:3]).at[3].set(eW1[4]).at[4].set(eW1[3])
            .at[5].set(eb1)).astype(jnp.bfloat16)
    eW2b = eW2.astype(jnp.bfloat16)
    eb2r = eb2.reshape(1, HID)
    eW3p = (jnp.zeros((HID, 8), jnp.float32).at[:, :5].set(eW3)
            ).astype(jnp.bfloat16)
    eb3p = jnp.zeros((1, 8), jnp.float32).at[0, :5].set(eb3).at[0, 5].set(1.0)

    nA = jnp.zeros((8, HID), jnp.float32).at[:5].set(nW1[:5])
    nB = jnp.zeros((FEAT, HID), jnp.float32).at[3:].set(nW1[5:])
    nb1r = nb1.reshape(1, HID)
    nb2r = nb2.reshape(1, HID)
    nb3r = nb3.reshape(1, FEAT)

    db1r = db1.reshape(1, HID)
    db2r = db2.reshape(1, HID)
    db3r = db3.reshape(1, HID)
    dW4p = jnp.zeros((HID, 8), jnp.float32).at[:, :3].set(dW4)
    db4p = jnp.zeros((1, 8), jnp.float32).at[0, :3].set(db4)

    x4 = jnp.concatenate(
        [x[:, :3], x[:, FEAT - 1:], jnp.zeros((N, 4), jnp.float32)], axis=1)

    gather_calls, scatter_calls = _sc_calls()
    for _ in range(3):
        ep0 = gather_calls[0](x4, row, col)
        ep1 = gather_calls[1](x4, row, col)
        ea0 = _edge_mlp(ep0, eW1p, eW2b, eb2r, eW3p, eb3p)
        ea1 = _edge_mlp(ep1, eW1p, eW2b, eb2r, eW3p, eb3p)
        agg_h0 = scatter_calls[0](ea0, col2d, zeros_init)
        agg2 = scatter_calls[1](ea1, col2d, agg_h0)
        x, x4 = _node_mlp(agg2, x, nA, nB, nb1r, nW2, nb2r, nW3, nb3r)

    out8 = _decoder(x, dW1, db1r, dW2, db2r, dW3, db3r, dW4p, db4p)
    return out8[:, :3]


# speedup vs baseline: 12.1572x; 1.0122x over previous
import functools

import jax
import jax.numpy as jnp
from jax import lax
from jax.experimental import pallas as pl
from jax.experimental.pallas import tpu as pltpu
from jax.experimental.pallas import tpu_sc as plsc

N = 10000
E = 320000
FEAT = 128
HID = 256

NC = 2
NS = 16
NW = NC * NS
PAN = 128
NPAN = E // PAN
HPAN = NPAN // 2
PPT = HPAN // NW
PREM = HPAN - PPT * NW
PCH = 13
ZR = N // NS

_SC_PARAMS = pltpu.CompilerParams(needs_layout_passes=False,
                                  use_tc_tiling_on_sc=False)


def _make_gather_body(abs_off):
  NCH = PPT // PCH

  def _gather_body(x4_hbm, row_hbm, col_hbm, out_hbm, tab_v, ridx_v, cidx_v,
                   obuf_v, isem, osem):
    cid = lax.axis_index("c")
    sid = lax.axis_index("s")
    wid = cid * NS + sid
    pltpu.sync_copy(x4_hbm, tab_v)

    def start_in(ch, buf):
        eb = (wid * PPT + ch * PCH + abs_off) * PAN
        ne = PCH * PAN
        pltpu.async_copy(row_hbm.at[pl.ds(eb, ne)],
                         ridx_v.at[buf], isem.at[buf])
        pltpu.async_copy(col_hbm.at[pl.ds(eb, ne)],
                         cidx_v.at[buf], isem.at[buf])

    def wait_in(ch, buf):
        eb = (wid * PPT + ch * PCH + abs_off) * PAN
        ne = PCH * PAN
        pltpu.make_async_copy(row_hbm.at[pl.ds(eb, ne)],
                              ridx_v.at[buf], isem.at[buf]).wait()
        pltpu.make_async_copy(col_hbm.at[pl.ds(eb, ne)],
                              cidx_v.at[buf], isem.at[buf]).wait()

    def compute(npan, buf):
        @plsc.parallel_loop(0, npan, unroll=4)
        def panel(jj):
            for g in range(8):
                o = jj * PAN + g * 16
                r = ridx_v[buf, pl.ds(o, 16)]
                cc = cidx_v[buf, pl.ds(o, 16)]
                d = []
                for col in range(4):
                    cv = jnp.full((16,), col, jnp.int32)
                    sv = plsc.load_gather(tab_v, [r, cv])
                    dv = plsc.load_gather(tab_v, [cc, cv])
                    d.append(dv - sv)
                    obuf_v[buf, jj, col, pl.ds(g * 16, 16)] = d[col]
                nsq = d[0] * d[0] + d[1] * d[1] + d[2] * d[2]
                i = plsc.bitcast(nsq, jnp.int32)
                i = 0x5F3759DF - lax.shift_right_logical(i, 1)
                y = plsc.bitcast(i, jnp.float32)
                for _ in range(3):
                    y = y * (1.5 - 0.5 * nsq * y * y)
                nrm = jnp.where(nsq > 0.0, nsq * y, 0.0)
                obuf_v[buf, jj, 4, pl.ds(g * 16, 16)] = nrm
                obuf_v[buf, jj, 5, pl.ds(g * 16, 16)] = jnp.full(
                    (16,), 1.0, jnp.float32)

    def start_out(ch, buf):
        pltpu.async_copy(obuf_v.at[buf],
                         out_hbm.at[pl.ds(wid * PPT + ch * PCH, PCH)],
                         osem.at[buf])

    def wait_out(ch, buf):
        pltpu.make_async_copy(obuf_v.at[buf],
                              out_hbm.at[pl.ds(wid * PPT + ch * PCH, PCH)],
                              osem.at[buf]).wait()

    start_in(0, 0)
    for ch in range(NCH):
        buf = ch % 2
        if ch + 1 < NCH:
            start_in(ch + 1, 1 - buf)
        wait_in(ch, buf)
        if ch >= 2:
            wait_out(ch - 2, buf)
        compute(PCH, buf)
        start_out(ch, buf)
    for ch in range(max(NCH - 2, 0), NCH):
        wait_out(ch, ch % 2)

    @pl.when(wid < PREM)
    def _():
        pbase = NW * PPT + wid
        eb = (pbase + abs_off) * PAN
        pltpu.sync_copy(row_hbm.at[pl.ds(eb, PAN)],
                        ridx_v.at[0, pl.ds(0, PAN)])
        pltpu.sync_copy(col_hbm.at[pl.ds(eb, PAN)],
                        cidx_v.at[0, pl.ds(0, PAN)])
        compute(1, 0)
        pltpu.sync_copy(obuf_v.at[0, pl.ds(0, 1)],
                        out_hbm.at[pl.ds(pbase, 1)])

  return _gather_body


def _make_scatter_body(abs_off):
  NCH = PPT // PCH

  def _scatter_body(ea_hbm, col2d_hbm, init_hbm, out_hbm, shared, ebuf_v,
                    ibuf_v, rbuf_v, isem, ssem):
    cid = lax.axis_index("c")
    sid = lax.axis_index("s")
    wid = cid * NS + sid
    pltpu.sync_copy(init_hbm.at[cid, pl.ds(sid * ZR, ZR)],
                    shared.at[pl.ds(sid * ZR, ZR)])
    zero16 = jnp.zeros((16,), jnp.float32)
    for rb in range(2):
        for g in range(8):
            rows = lax.iota(jnp.int32, 16) + g * 16
            plsc.store_scatter(rbuf_v.at[rb],
                               [rows, jnp.full((16,), 6, jnp.int32)], zero16)
            plsc.store_scatter(rbuf_v.at[rb],
                               [rows, jnp.full((16,), 7, jnp.int32)], zero16)
    plsc.subcore_barrier()

    def start_in(ch, buf):
        pb = wid * PPT + ch * PCH
        pltpu.async_copy(ea_hbm.at[pl.ds(pb, PCH)], ebuf_v.at[buf],
                         isem.at[buf])
        pltpu.async_copy(col2d_hbm.at[pl.ds(pb + abs_off, PCH)],
                         ibuf_v.at[buf], isem.at[buf])

    def wait_in(ch, buf):
        pb = wid * PPT + ch * PCH
        pltpu.make_async_copy(ea_hbm.at[pl.ds(pb, PCH)], ebuf_v.at[buf],
                              isem.at[buf]).wait()
        pltpu.make_async_copy(col2d_hbm.at[pl.ds(pb + abs_off, PCH)],
                              ibuf_v.at[buf], isem.at[buf]).wait()

    def build(buf, jj, rb):
        for g in range(8):
            rows = lax.iota(jnp.int32, 16) + g * 16
            for c in range(6):
                v = ebuf_v[buf, jj, c, pl.ds(g * 16, 16)]
                plsc.store_scatter(rbuf_v.at[rb],
                                   [rows, jnp.full((16,), c, jnp.int32)], v)

    def drain(rb):
        pltpu.make_async_copy(rbuf_v.at[rb], shared.at[pl.ds(0, PAN)],
                              ssem.at[rb]).wait()

    start_in(0, 0)
    for ch in range(NCH):
        buf = ch % 2
        if ch + 1 < NCH:
            start_in(ch + 1, 1 - buf)
        wait_in(ch, buf)
        for jj in range(PCH):
            rb = jj % 2
            if jj >= 2:
                drain(rb)
            build(buf, jj, rb)
            pltpu.async_copy(rbuf_v.at[rb], shared.at[ibuf_v.at[buf, jj]],
                             ssem.at[rb], add=True)
        drain(0)
        drain(1)

    @pl.when(wid < PREM)
    def _():
        pbase = NW * PPT + wid
        pltpu.sync_copy(ea_hbm.at[pl.ds(pbase, 1)], ebuf_v.at[0, pl.ds(0, 1)])
        pltpu.sync_copy(col2d_hbm.at[pl.ds(pbase + abs_off, 1)],
                        ibuf_v.at[0, pl.ds(0, 1)])
        build(0, 0, 0)
        pltpu.sync_copy(rbuf_v.at[0], shared.at[ibuf_v.at[0, 0]], add=True)

    plsc.subcore_barrier()
    pltpu.sync_copy(shared.at[pl.ds(sid * ZR, ZR)],
                    out_hbm.at[cid, pl.ds(sid * ZR, ZR)])

  return _scatter_body


@functools.lru_cache(maxsize=None)
def _sc_calls():
    mesh = plsc.VectorSubcoreMesh(core_axis_name="c", subcore_axis_name="s",
                                  num_cores=NC, num_subcores=NS)
    gather_calls = []
    scatter_calls = []
    for half in range(2):
        gather_calls.append(pl.kernel(
            _make_gather_body(half * HPAN),
            out_type=jax.ShapeDtypeStruct((HPAN, 8, PAN), jnp.float32),
            mesh=mesh,
            compiler_params=_SC_PARAMS,
            scratch_types=[
                pltpu.VMEM((N, 8), jnp.float32),
                pltpu.VMEM((2, PCH * PAN), jnp.int32),
                pltpu.VMEM((2, PCH * PAN), jnp.int32),
                pltpu.VMEM((2, PCH, 8, PAN), jnp.float32),
                pltpu.SemaphoreType.DMA((2,)),
                pltpu.SemaphoreType.DMA((2,)),
            ],
        ))
        scatter_calls.append(pl.kernel(
            _make_scatter_body(half * HPAN),
            out_type=jax.ShapeDtypeStruct((NC, N, 8), jnp.float32),
            mesh=mesh,
            compiler_params=_SC_PARAMS,
            scratch_types=[
                pltpu.VMEM_SHARED((N, 8), jnp.float32),
                pltpu.VMEM((2, PCH, 8, PAN), jnp.float32),
                pltpu.VMEM((2, PCH, PAN), jnp.int32),
                pltpu.VMEM((2, PAN, 8), jnp.float32),
                pltpu.SemaphoreType.DMA((2,)),
                pltpu.SemaphoreType.DMA((2,)),
            ],
        ))
    return gather_calls, scatter_calls


BP = 50
BLK_N = 2000


def _edge_mlp_body(e_ref, w1_ref, w2_ref, b2_ref, w3_ref,
                   b3_ref, o_ref):
    e3 = e_ref[...]
    e = jnp.transpose(e3, (0, 2, 1)).reshape(BP * PAN, 8)
    h = jnp.maximum(
        jnp.dot(e.astype(jnp.bfloat16), w1_ref[...],
                preferred_element_type=jnp.float32), 0.0)
    h = jnp.maximum(
        jnp.dot(h.astype(jnp.bfloat16), w2_ref[...],
                preferred_element_type=jnp.float32)
        + b2_ref[...], 0.0)
    o = (jnp.dot(h.astype(jnp.bfloat16), w3_ref[...],
                 preferred_element_type=jnp.float32)
         + b3_ref[...])
    o_ref[...] = jnp.transpose(o.reshape(BP, PAN, 8), (0, 2, 1))


_edge_mlp = pl.pallas_call(
    _edge_mlp_body,
    grid=(HPAN // BP,),
    in_specs=[
        pl.BlockSpec((BP, 8, PAN), lambda i: (i, 0, 0)),
        pl.BlockSpec((8, HID), lambda i: (0, 0)),
        pl.BlockSpec((HID, HID), lambda i: (0, 0)),
        pl.BlockSpec((1, HID), lambda i: (0, 0)),
        pl.BlockSpec((HID, 8), lambda i: (0, 0)),
        pl.BlockSpec((1, 8), lambda i: (0, 0)),
    ],
    out_specs=pl.BlockSpec((BP, 8, PAN), lambda i: (i, 0, 0)),
    out_shape=jax.ShapeDtypeStruct((HPAN, 8, PAN), jnp.float32),
    compiler_params=pltpu.CompilerParams(dimension_semantics=("parallel",)),
)


def _node_mlp_body(agg_ref, x_ref, a_ref, bmat_ref, b1_ref, w2_ref, b2_ref,
                   w3_ref, b3_ref, xo_ref, x4o_ref):
    agg2 = agg_ref[...]
    agg = agg2[0] + agg2[1]
    cnt = agg[:, 5:6]
    inv = 1.0 / jnp.maximum(cnt, 1.0)
    m = agg * inv
    x = x_ref[...]
    h = jnp.maximum(
        jnp.dot(m, a_ref[...], preferred_element_type=jnp.float32)
        + jnp.dot(x, bmat_ref[...], preferred_element_type=jnp.float32)
        + b1_ref[...], 0.0)
    h = jnp.maximum(
        jnp.dot(h, w2_ref[...], preferred_element_type=jnp.float32)
        + b2_ref[...], 0.0)
    res = (jnp.dot(h, w3_ref[...], preferred_element_type=jnp.float32)
           + b3_ref[...])
    xn = x + jnp.maximum(res, 0.0)
    xo_ref[...] = xn
    x4o_ref[...] = jnp.concatenate(
        [xn[:, 0:3], xn[:, FEAT - 1:FEAT],
         jnp.zeros((xn.shape[0], 4), jnp.float32)], axis=1)


_node_mlp = pl.pallas_call(
    _node_mlp_body,
    grid=(N // BLK_N,),
    in_specs=[
        pl.BlockSpec((NC, BLK_N, 8), lambda i: (0, i, 0)),
        pl.BlockSpec((BLK_N, FEAT), lambda i: (i, 0)),
        pl.BlockSpec((8, HID), lambda i: (0, 0)),
        pl.BlockSpec((FEAT, HID), lambda i: (0, 0)),
        pl.BlockSpec((1, HID), lambda i: (0, 0)),
        pl.BlockSpec((HID, HID), lambda i: (0, 0)),
        pl.BlockSpec((1, HID), lambda i: (0, 0)),
        pl.BlockSpec((HID, FEAT), lambda i: (0, 0)),
        pl.BlockSpec((1, FEAT), lambda i: (0, 0)),
    ],
    out_specs=[
        pl.BlockSpec((BLK_N, FEAT), lambda i: (i, 0)),
        pl.BlockSpec((BLK_N, 8), lambda i: (i, 0)),
    ],
    out_shape=[
        jax.ShapeDtypeStruct((N, FEAT), jnp.float32),
        jax.ShapeDtypeStruct((N, 8), jnp.float32),
    ],
    compiler_params=pltpu.CompilerParams(dimension_semantics=("parallel",)),
)


def _decoder_body(x_ref, w1_ref, b1_ref, w2_ref, b2_ref, w3_ref, b3_ref,
                  w4_ref, b4_ref, o_ref):
    h = jnp.maximum(
        jnp.dot(x_ref[...], w1_ref[...], preferred_element_type=jnp.float32)
        + b1_ref[...], 0.0)
    h = jnp.maximum(
        jnp.dot(h, w2_ref[...], preferred_element_type=jnp.float32)
        + b2_ref[...], 0.0)
    h = jnp.maximum(
        jnp.dot(h, w3_ref[...], preferred_element_type=jnp.float32)
        + b3_ref[...], 0.0)
    o_ref[...] = (jnp.dot(h, w4_ref[...], preferred_element_type=jnp.float32)
                  + b4_ref[...])


_decoder = pl.pallas_call(
    _decoder_body,
    grid=(N // BLK_N,),
    in_specs=[
        pl.BlockSpec((BLK_N, FEAT), lambda i: (i, 0)),
        pl.BlockSpec((FEAT, HID), lambda i: (0, 0)),
        pl.BlockSpec((1, HID), lambda i: (0, 0)),
        pl.BlockSpec((HID, HID), lambda i: (0, 0)),
        pl.BlockSpec((1, HID), lambda i: (0, 0)),
        pl.BlockSpec((HID, HID), lambda i: (0, 0)),
        pl.BlockSpec((1, HID), lambda i: (0, 0)),
        pl.BlockSpec((HID, 8), lambda i: (0, 0)),
        pl.BlockSpec((1, 8), lambda i: (0, 0)),
    ],
    out_specs=pl.BlockSpec((BLK_N, 8), lambda i: (i, 0)),
    out_shape=jax.ShapeDtypeStruct((N, 8), jnp.float32),
    compiler_params=pltpu.CompilerParams(dimension_semantics=("parallel",)),
)


def kernel(x, edge_index, mode, eW1, eb1, eW2, eb2, eW3, eb3, nW1, nb1, nW2,
           nb2, nW3, nb3, dW1, db1, dW2, db2, dW3, db3, dW4, db4):
    del mode
    row = edge_index[0]
    col = edge_index[1]
    col2d = col.reshape(NPAN, PAN)
    zeros_init = jnp.zeros((NC, N, 8), jnp.float32)

    eW1p = (jnp.zeros((8, HID), jnp.float32)
            .at[:3].set(eW1[:3]).at[3].set(eW1[4]).at[4].set(eW1[3])
            .at[5].set(eb1)).astype(jnp.bfloat16)
    eW2b = eW2.astype(jnp.bfloat16)
    eb2r = eb2.reshape(1, HID)
    eW3p = (jnp.zeros((HID, 8), jnp.float32).at[:, :5].set(eW3)
            ).astype(jnp.bfloat16)
    eb3p = jnp.zeros((1, 8), jnp.float32).at[0, :5].set(eb3).at[0, 5].set(1.0)

    nA = jnp.zeros((8, HID), jnp.float32).at[:5].set(nW1[:5])
    nB = jnp.zeros((FEAT, HID), jnp.float32).at[3:].set(nW1[5:])
    nb1r = nb1.reshape(1, HID)
    nb2r = nb2.reshape(1, HID)
    nb3r = nb3.reshape(1, FEAT)

    db1r = db1.reshape(1, HID)
    db2r = db2.reshape(1, HID)
    db3r = db3.reshape(1, HID)
    dW4p = jnp.zeros((HID, 8), jnp.float32).at[:, :3].set(dW4)
    db4p = jnp.zeros((1, 8), jnp.float32).at[0, :3].set(db4)

    x4 = jnp.concatenate(
        [x[:, :3], x[:, FEAT - 1:], jnp.zeros((N, 4), jnp.float32)], axis=1)

    gather_calls, scatter_calls = _sc_calls()
    for _ in range(3):
        ep0 = gather_calls[0](x4, row, col)
        ep1 = gather_calls[1](x4, row, col)
        ea0 = _edge_mlp(ep0, eW1p, eW2b, eb2r, eW3p, eb3p)
        ea1 = _edge_mlp(ep1, eW1p, eW2b, eb2r, eW3p, eb3p)
        agg_h0 = scatter_calls[0](ea0, col2d, zeros_init)
        agg2 = scatter_calls[1](ea1, col2d, agg_h0)
        x, x4 = _node_mlp(agg2, x, nA, nB, nb1r, nW2, nb2r, nW3, nb3r)

    out8 = _decoder(x, dW1, db1r, dW2, db2r, dW3, db3r, dW4p, db4p)
    return out8[:, :3]
